# baseline (speedup 1.0000x reference)
import functools

import jax
import jax.numpy as jnp
from jax import lax
from jax.experimental import pallas as pl
from jax.experimental.pallas import tpu as pltpu
from jax.experimental.pallas import tpu_sc as plsc

N_NODES_C = 10000
N_EDGES_C = 320000
EB = 2000
NB = 2000

SC_NC = 2
SC_NS = 16
SC_NW = SC_NC * SC_NS
SC_C = 40
NPAD = 10240


def _mlp_ln_val(x, w1, b1, w2, b2, g, o):
    h = jnp.maximum(jnp.dot(x, w1, preferred_element_type=jnp.float32) + b1, 0.0)
    y = jnp.dot(h, w2, preferred_element_type=jnp.float32) + b2
    m = jnp.mean(y, axis=-1, keepdims=True)
    v = jnp.mean((y - m) ** 2, axis=-1, keepdims=True)
    return g * (y - m) / jnp.sqrt(v + 1e-5) + o


def _full(shape):
    return pl.BlockSpec(shape, lambda i: (0,) * len(shape))


def _rows(bs, d):
    return pl.BlockSpec((bs, d), lambda i: (i, 0))


def _p6(p):
    return (p["w1"], p["b1"][None, :], p["w2"], p["b2"][None, :],
            p["g"][None, :], p["o"][None, :])


def _p6_specs(in_dim, hid, out_dim):
    return [_full((in_dim, hid)), _full((1, hid)), _full((hid, out_dim)),
            _full((1, out_dim)), _full((1, out_dim)), _full((1, out_dim))]


def _node_encode_body(x, w1, b1, w2, b2, g, o, ws, wr, n_out, gs_out, gr_out):
    n = _mlp_ln_val(x[...], w1[...], b1[...], w2[...], b2[...], g[...], o[...])
    n_out[...] = n
    gs_out[...] = jnp.dot(
        n, ws[...], preferred_element_type=jnp.float32).astype(jnp.bfloat16)
    gr_out[...] = jnp.dot(
        n, wr[...], preferred_element_type=jnp.float32).astype(jnp.bfloat16)


def _node_encode(node_features, enc_p, w1s, w1r):
    n, d = node_features.shape
    lat = enc_p["w2"].shape[1]
    grid = (n // NB,)
    return pl.pallas_call(
        _node_encode_body,
        grid=grid,
        in_specs=[_rows(NB, d)] + _p6_specs(d, enc_p["w1"].shape[1], lat)
                 + [_full((lat, lat)), _full((lat, lat))],
        out_specs=[_rows(NB, lat)] * 3,
        out_shape=[jax.ShapeDtypeStruct((n, lat), jnp.float32)]
                  + [jax.ShapeDtypeStruct((n, lat), jnp.bfloat16)] * 2,
    )(node_features, *_p6(enc_p), w1s, w1r)


def _edge_step0_body(feat, gsum, ew1, eb1, ew2, eb2, eg, eo,
                     w1e, b1, w2, b2, g, o, ne_out, e1_out):
    e0 = _mlp_ln_val(feat[...], ew1[...], eb1[...], ew2[...], eb2[...],
                     eg[...], eo[...])
    h = jnp.maximum(
        jnp.dot(e0, w1e[...], preferred_element_type=jnp.float32)
        + gsum[...].astype(jnp.float32) + b1[...], 0.0)
    y = jnp.dot(h, w2[...], preferred_element_type=jnp.float32) + b2[...]
    m = jnp.mean(y, axis=-1, keepdims=True)
    v = jnp.mean((y - m) ** 2, axis=-1, keepdims=True)
    ne = g[...] * (y - m) / jnp.sqrt(v + 1e-5) + o[...]
    ne_out[...] = ne
    e1_out[...] = (e0 + ne).astype(jnp.bfloat16)


def _edge_step0(edge_features, gsum, enc_p, w1e, proc_p):
    e, d = edge_features.shape
    hid = enc_p["w1"].shape[1]
    lat = enc_p["w2"].shape[1]
    grid = (e // EB,)
    return pl.pallas_call(
        _edge_step0_body,
        grid=grid,
        in_specs=[_rows(EB, d), _rows(EB, lat)]
                 + _p6_specs(d, hid, lat)
                 + [_full((lat, hid)), _full((1, hid)), _full((hid, lat)),
                    _full((1, lat)), _full((1, lat)), _full((1, lat))],
        out_specs=[_rows(EB, lat)] * 2,
        out_shape=[jax.ShapeDtypeStruct((e, lat), jnp.float32),
                   jax.ShapeDtypeStruct((e, lat), jnp.bfloat16)],
    )(edge_features, gsum, *_p6(enc_p), w1e, proc_p["b1"][None, :],
      proc_p["w2"], proc_p["b2"][None, :], proc_p["g"][None, :],
      proc_p["o"][None, :])


def _edge_step1_body(ecur, gsum, w1e, b1, w2, b2, g, o, ne_out):
    h = jnp.maximum(
        jnp.dot(ecur[...].astype(jnp.float32), w1e[...],
                preferred_element_type=jnp.float32)
        + gsum[...].astype(jnp.float32) + b1[...], 0.0)
    y = jnp.dot(h, w2[...], preferred_element_type=jnp.float32) + b2[...]
    m = jnp.mean(y, axis=-1, keepdims=True)
    v = jnp.mean((y - m) ** 2, axis=-1, keepdims=True)
    ne_out[...] = g[...] * (y - m) / jnp.sqrt(v + 1e-5) + o[...]


def _edge_step1(ecur, gsum, w1e, proc_p):
    e, lat = ecur.shape
    hid = proc_p["w2"].shape[0]
    grid = (e // EB,)
    return pl.pallas_call(
        _edge_step1_body,
        grid=grid,
        in_specs=[_rows(EB, lat), _rows(EB, lat), _full((lat, hid)),
                  _full((1, hid)), _full((hid, lat)), _full((1, lat)),
                  _full((1, lat)), _full((1, lat))],
        out_specs=[_rows(EB, lat)],
        out_shape=[jax.ShapeDtypeStruct((e, lat), jnp.float32)],
    )(ecur, gsum, w1e, proc_p["b1"][None, :], proc_p["w2"],
      proc_p["b2"][None, :], proc_p["g"][None, :], proc_p["o"][None, :])[0]


def _node_step0_body(nodes, agg, w1n, w1a, b1, w2, b2, g, o, ws, wr,
                     n_out, gs_out, gr_out):
    a = agg[...]
    h = jnp.maximum(
        jnp.dot(nodes[...], w1n[...], preferred_element_type=jnp.float32)
        + jnp.dot(a, w1a[...], preferred_element_type=jnp.float32)
        + b1[...], 0.0)
    y = jnp.dot(h, w2[...], preferred_element_type=jnp.float32) + b2[...]
    m = jnp.mean(y, axis=-1, keepdims=True)
    v = jnp.mean((y - m) ** 2, axis=-1, keepdims=True)
    n1 = nodes[...] + g[...] * (y - m) / jnp.sqrt(v + 1e-5) + o[...]
    n_out[...] = n1
    gs_out[...] = jnp.dot(
        n1, ws[...], preferred_element_type=jnp.float32).astype(jnp.bfloat16)
    gr_out[...] = jnp.dot(
        n1, wr[...], preferred_element_type=jnp.float32).astype(jnp.bfloat16)


def _node_step0(nodes, agg, w1n, w1a, proc_p, w1s, w1r):
    n, lat = nodes.shape
    hid = proc_p["w2"].shape[0]
    grid = (n // NB,)
    return pl.pallas_call(
        _node_step0_body,
        grid=grid,
        in_specs=[_rows(NB, lat), _rows(NB, lat),
                  _full((lat, hid)), _full((lat, hid)), _full((1, hid)),
                  _full((hid, lat)), _full((1, lat)), _full((1, lat)),
                  _full((1, lat)), _full((lat, lat)), _full((lat, lat))],
        out_specs=[_rows(NB, lat)] * 3,
        out_shape=[jax.ShapeDtypeStruct((n, lat), jnp.float32)]
                  + [jax.ShapeDtypeStruct((n, lat), jnp.bfloat16)] * 2,
    )(nodes, agg, w1n, w1a, proc_p["b1"][None, :], proc_p["w2"],
      proc_p["b2"][None, :], proc_p["g"][None, :], proc_p["o"][None, :],
      w1s, w1r)


def _node_step1_decode_body(nodes, agg, w1n, w1a, b1, w2, b2, g, o,
                            pw1, pb1, pw2, pb2, pg, po,
                            qw1, qb1, qw2, qb2, qg, qo,
                            lw1, lb1, lw2, lb2, lg, lo,
                            emb_out, proj_out, pred_out, logit_out):
    a = agg[...]
    h = jnp.maximum(
        jnp.dot(nodes[...], w1n[...], preferred_element_type=jnp.float32)
        + jnp.dot(a, w1a[...], preferred_element_type=jnp.float32)
        + b1[...], 0.0)
    y = jnp.dot(h, w2[...], preferred_element_type=jnp.float32) + b2[...]
    m = jnp.mean(y, axis=-1, keepdims=True)
    v = jnp.mean((y - m) ** 2, axis=-1, keepdims=True)
    n2 = nodes[...] + g[...] * (y - m) / jnp.sqrt(v + 1e-5) + o[...]
    emb_out[...] = n2
    proj = _mlp_ln_val(n2, pw1[...], pb1[...], pw2[...], pb2[...], pg[...],
                       po[...])
    proj_out[...] = proj
    pred_out[...] = _mlp_ln_val(proj, qw1[...], qb1[...], qw2[...], qb2[...],
                                qg[...], qo[...])
    logit_out[...] = _mlp_ln_val(n2, lw1[...], lb1[...], lw2[...], lb2[...],
                                 lg[...], lo[...])


def _node_step1_decode(nodes, agg, w1n, w1a, proc_p, proj_p, pred_p, log_p):
    n, lat = nodes.shape
    hid = proc_p["w2"].shape[0]
    ncls = log_p["w2"].shape[1]
    grid = (n // NB,)
    return pl.pallas_call(
        _node_step1_decode_body,
        grid=grid,
        in_specs=[_rows(NB, lat), _rows(NB, lat),
                  _full((lat, hid)), _full((lat, hid)), _full((1, hid)),
                  _full((hid, lat)), _full((1, lat)), _full((1, lat)),
                  _full((1, lat))]
                 + _p6_specs(lat, hid, lat) + _p6_specs(lat, hid, lat)
                 + _p6_specs(lat, hid, ncls),
        out_specs=[_rows(NB, lat)] * 3 + [_rows(NB, ncls)],
        out_shape=[jax.ShapeDtypeStruct((n, lat), jnp.float32)] * 3
                  + [jax.ShapeDtypeStruct((n, ncls), jnp.float32)],
    )(nodes, agg, w1n, w1a, proc_p["b1"][None, :], proc_p["w2"],
      proc_p["b2"][None, :], proc_p["g"][None, :], proc_p["o"][None, :],
      *_p6(proj_p), *_p6(pred_p), *_p6(log_p))


def _sc_mesh():
    return plsc.VectorSubcoreMesh(core_axis_name="c", subcore_axis_name="s",
                                  num_cores=SC_NC, num_subcores=SC_NS)


def _gather_sum_body(gs_hbm, gr_hbm, snd2_hbm, rcv2_hbm, out_hbm,
                     idx_s, idx_r, ra0, rb0, ro0, ra1, rb1, ro1,
                     sg0, sg1, sw0, sw1):
    nchunk = idx_s.shape[0]
    lat = ra0.shape[1]
    wid = lax.axis_index("s") * SC_NC + lax.axis_index("c")
    base0 = wid * (nchunk * SC_C)
    ra = (ra0, ra1)
    rb = (rb0, rb1)
    ro = (ro0, ro1)
    sg = (sg0, sg1)
    sw = (sw0, sw1)

    pltpu.sync_copy(snd2_hbm.at[pl.ds(wid * nchunk, nchunk)], idx_s)
    pltpu.sync_copy(rcv2_hbm.at[pl.ds(wid * nchunk, nchunk)], idx_r)

    def issue_gather(j, p):
        pltpu.async_copy(gs_hbm.at[idx_s.at[j]], ra[p], sg[p])
        pltpu.async_copy(gr_hbm.at[idx_r.at[j]], rb[p], sg[p])

    def wait_gather(j, p):
        pltpu.make_async_copy(gs_hbm.at[idx_s.at[j]], ra[p], sg[p]).wait()
        pltpu.make_async_copy(gr_hbm.at[idx_r.at[j]], rb[p], sg[p]).wait()

    def do_add(p):
        def addrow(r, c2):
            for j in range(lat // 32):
                sl = pl.ds(j * 32, 32)
                ro[p][r, sl] = ra[p][r, sl] + rb[p][r, sl]
            return c2
        lax.fori_loop(0, SC_C, addrow, 0)

    def issue_write(j, p):
        pltpu.async_copy(ro[p], out_hbm.at[pl.ds(base0 + j * SC_C, SC_C)],
                         sw[p])

    def wait_write(j, p):
        pltpu.make_async_copy(
            ro[p], out_hbm.at[pl.ds(base0 + j * SC_C, SC_C)], sw[p]).wait()

    issue_gather(0, 0)
    issue_gather(1, 1)
    for p in (0, 1):
        wait_gather(p, p)
        do_add(p)
        issue_write(p, p)
        issue_gather(p + 2, p)

    def pair(t, carry):
        for p in (0, 1):
            jj = 2 * t + p
            wait_gather(jj, p)
            wait_write(jj - 2, p)
            do_add(p)
            issue_write(jj, p)
            issue_gather(jj + 2, p)
        return carry

    lax.fori_loop(1, (nchunk - 2) // 2, pair, 0)

    for p in (0, 1):
        jj = nchunk - 2 + p
        wait_gather(jj, p)
        wait_write(jj - 2, p)
        do_add(p)
        issue_write(jj, p)
    for p in (0, 1):
        wait_write(nchunk - 2 + p, p)


def _gather_sum(gs, gr, snd2, rcv2):
    n_edges = snd2.shape[0] * snd2.shape[1]
    lat = gs.shape[1]
    nchunk = n_edges // (SC_NW * SC_C)
    f = pl.kernel(
        _gather_sum_body,
        out_type=jax.ShapeDtypeStruct((n_edges, lat), jnp.bfloat16),
        mesh=_sc_mesh(),
        compiler_params=pltpu.CompilerParams(use_tc_tiling_on_sc=False),
        scratch_types=[
            pltpu.VMEM((nchunk, SC_C), jnp.int32),
            pltpu.VMEM((nchunk, SC_C), jnp.int32),
            pltpu.VMEM((SC_C, lat), jnp.bfloat16),
            pltpu.VMEM((SC_C, lat), jnp.bfloat16),
            pltpu.VMEM((SC_C, lat), jnp.bfloat16),
            pltpu.VMEM((SC_C, lat), jnp.bfloat16),
            pltpu.VMEM((SC_C, lat), jnp.bfloat16),
            pltpu.VMEM((SC_C, lat), jnp.bfloat16),
            pltpu.SemaphoreType.DMA,
            pltpu.SemaphoreType.DMA,
            pltpu.SemaphoreType.DMA,
            pltpu.SemaphoreType.DMA,
        ],
    )
    return f(gs, gr, snd2, rcv2)


def _segment_sum_body(ne_hbm, rcv2_hbm, out_hbm, shared, zbuf, idx_v,
                      rv0, rv1, sl0, sl1):
    lat = ne_hbm.shape[1]
    hc = lat // SC_NC
    nchunk = idx_v.shape[0]
    cid = lax.axis_index("c")
    sid = lax.axis_index("s")
    base0 = sid * (nchunk * SC_C)
    rv = (rv0, rv1)
    sl = (sl0, sl1)
    rows_per_sub = NPAD // SC_NS
    zero = jnp.zeros((16,), jnp.float32)

    def zrow(r, carry):
        for j in range(hc // 16):
            zbuf[r, pl.ds(j * 16, 16)] = zero
        return carry

    lax.fori_loop(0, rows_per_sub, zrow, 0)
    pltpu.sync_copy(zbuf, shared.at[pl.ds(sid * rows_per_sub, rows_per_sub)])
    pltpu.sync_copy(rcv2_hbm.at[pl.ds(sid * nchunk, nchunk)], idx_v)
    plsc.subcore_barrier()

    def issue_load(j, p):
        pltpu.async_copy(
            ne_hbm.at[pl.ds(base0 + j * SC_C, SC_C), pl.ds(cid * hc, hc)],
            rv[p], sl[p])

    def wait_load(j, p):
        pltpu.make_async_copy(
            ne_hbm.at[pl.ds(base0 + j * SC_C, SC_C), pl.ds(cid * hc, hc)],
            rv[p], sl[p]).wait()

    def scat(j, p):
        pltpu.sync_copy(rv[p], shared.at[idx_v.at[j]], add=True)

    issue_load(0, 0)
    issue_load(1, 1)
    for p in (0, 1):
        wait_load(p, p)
        scat(p, p)
        issue_load(p + 2, p)

    def pair(t, carry):
        for p in (0, 1):
            jj = 2 * t + p
            wait_load(jj, p)
            scat(jj, p)
            issue_load(jj + 2, p)
        return carry

    lax.fori_loop(1, (nchunk - 2) // 2, pair, 0)

    for p in (0, 1):
        jj = nchunk - 2 + p
        wait_load(jj, p)
        scat(jj, p)

    plsc.subcore_barrier()
    pltpu.sync_copy(shared.at[pl.ds(sid * rows_per_sub, rows_per_sub)],
                    out_hbm.at[pl.ds(sid * rows_per_sub, rows_per_sub),
                               pl.ds(cid * hc, hc)])


def _segment_sum(ne, rcv2):
    lat = ne.shape[1]
    hc = lat // SC_NC
    n_edges = ne.shape[0]
    nchunk = n_edges // (SC_NS * SC_C)
    f = pl.kernel(
        _segment_sum_body,
        out_type=jax.ShapeDtypeStruct((NPAD, lat), jnp.float32),
        mesh=_sc_mesh(),
        compiler_params=pltpu.CompilerParams(use_tc_tiling_on_sc=False),
        scratch_types=[
            pltpu.VMEM_SHARED((NPAD, hc), jnp.float32),
            pltpu.VMEM((NPAD // SC_NS, hc), jnp.float32),
            pltpu.VMEM((nchunk, SC_C), jnp.int32),
            pltpu.VMEM((SC_C, hc), jnp.float32),
            pltpu.VMEM((SC_C, hc), jnp.float32),
            pltpu.SemaphoreType.DMA,
            pltpu.SemaphoreType.DMA,
        ],
    )
    return f(ne, rcv2)


def kernel(node_features, edge_features, senders, receivers, params):
    p = params
    n_nodes = node_features.shape[0]

    def w1_split3(pp):
        lat = pp["w2"].shape[0]
        return (pp["w1"][:lat], pp["w1"][lat:2 * lat], pp["w1"][2 * lat:])

    def w1_split2(pp):
        lat = pp["w2"].shape[0]
        return (pp["w1"][:lat], pp["w1"][lat:])

    e_w1e0, e_w1s0, e_w1r0 = w1_split3(p["edge_proc_0"])
    e_w1e1, e_w1s1, e_w1r1 = w1_split3(p["edge_proc_1"])
    n_w1n0, n_w1a0 = w1_split2(p["node_proc_0"])
    n_w1n1, n_w1a1 = w1_split2(p["node_proc_1"])

    snd2 = senders.reshape(-1, SC_C)
    rcv2 = receivers.reshape(-1, SC_C)

    nodes0, gs0, gr0 = _node_encode(node_features, p["node_encoder"],
                                    e_w1s0, e_w1r0)
    gsum0 = _gather_sum(gs0, gr0, snd2, rcv2)
    ne0, e1 = _edge_step0(edge_features, gsum0, p["edge_encoder"], e_w1e0,
                          p["edge_proc_0"])
    agg0 = _segment_sum(ne0, rcv2)
    nodes1, gs1, gr1 = _node_step0(nodes0, agg0, n_w1n0, n_w1a0,
                                   p["node_proc_0"], e_w1s1, e_w1r1)
    gsum1 = _gather_sum(gs1, gr1, snd2, rcv2)
    ne1 = _edge_step1(e1, gsum1, e_w1e1, p["edge_proc_1"])
    agg1 = _segment_sum(ne1, rcv2)
    emb, proj, pred, logit = _node_step1_decode(
        nodes1, agg1, n_w1n1, n_w1a1, p["node_proc_1"],
        p["projector"], p["predictor"], p["logits_decoder"])
    return (emb, proj, pred, logit)


# speedup vs baseline: 1.3437x; 1.3437x over previous
import functools

import jax
import jax.numpy as jnp
from jax import lax
from jax.experimental import pallas as pl
from jax.experimental.pallas import tpu as pltpu
from jax.experimental.pallas import tpu_sc as plsc

N_NODES_C = 10000
N_EDGES_C = 320000
EB = 2000
NB = 2000

SC_NC = 2
SC_NS = 16
SC_NW = SC_NC * SC_NS
SC_C = 40
NPAD = 10240


def _mlp_ln_val(x, w1, b1, w2, b2, g, o):
    h = jnp.maximum(jnp.dot(x, w1, preferred_element_type=jnp.float32) + b1, 0.0)
    y = jnp.dot(h, w2, preferred_element_type=jnp.float32) + b2
    m = jnp.mean(y, axis=-1, keepdims=True)
    v = jnp.mean((y - m) ** 2, axis=-1, keepdims=True)
    return g * (y - m) / jnp.sqrt(v + 1e-5) + o


def _full(shape):
    return pl.BlockSpec(shape, lambda i: (0,) * len(shape))


def _rows(bs, d):
    return pl.BlockSpec((bs, d), lambda i: (i, 0))


def _p6(p):
    return (p["w1"], p["b1"][None, :], p["w2"], p["b2"][None, :],
            p["g"][None, :], p["o"][None, :])


def _p6_specs(in_dim, hid, out_dim):
    return [_full((in_dim, hid)), _full((1, hid)), _full((hid, out_dim)),
            _full((1, out_dim)), _full((1, out_dim)), _full((1, out_dim))]


def _node_encode_body(x, w1, b1, w2, b2, g, o, ws, wr, n_out, gs_out, gr_out):
    n = _mlp_ln_val(x[...], w1[...], b1[...], w2[...], b2[...], g[...], o[...])
    n_out[...] = n
    gs_out[...] = jnp.dot(n, ws[...], preferred_element_type=jnp.float32)
    gr_out[...] = jnp.dot(n, wr[...], preferred_element_type=jnp.float32)


def _node_encode(node_features, enc_p, w1s, w1r):
    n, d = node_features.shape
    lat = enc_p["w2"].shape[1]
    grid = (n // NB,)
    return pl.pallas_call(
        _node_encode_body,
        grid=grid,
        in_specs=[_rows(NB, d)] + _p6_specs(d, enc_p["w1"].shape[1], lat)
                 + [_full((lat, lat)), _full((lat, lat))],
        out_specs=[_rows(NB, lat)] * 3,
        out_shape=[jax.ShapeDtypeStruct((n, lat), jnp.float32)] * 3,
    )(node_features, *_p6(enc_p), w1s, w1r)


def _edge_step0_body(feat, gsum, ew1, eb1, ew2, eb2, eg, eo,
                     w1e, b1, w2, b2, g, o, ne_out, e1_out):
    e0 = _mlp_ln_val(feat[...], ew1[...], eb1[...], ew2[...], eb2[...],
                     eg[...], eo[...])
    h = jnp.maximum(
        jnp.dot(e0, w1e[...], preferred_element_type=jnp.float32)
        + gsum[...] + b1[...], 0.0)
    y = jnp.dot(h, w2[...], preferred_element_type=jnp.float32) + b2[...]
    m = jnp.mean(y, axis=-1, keepdims=True)
    v = jnp.mean((y - m) ** 2, axis=-1, keepdims=True)
    ne = g[...] * (y - m) / jnp.sqrt(v + 1e-5) + o[...]
    ne_out[...] = ne
    e1_out[...] = (e0 + ne).astype(jnp.bfloat16)


def _edge_step0(edge_features, gsum, enc_p, w1e, proc_p):
    e, d = edge_features.shape
    hid = enc_p["w1"].shape[1]
    lat = enc_p["w2"].shape[1]
    grid = (e // EB,)
    return pl.pallas_call(
        _edge_step0_body,
        grid=grid,
        in_specs=[_rows(EB, d), _rows(EB, lat)]
                 + _p6_specs(d, hid, lat)
                 + [_full((lat, hid)), _full((1, hid)), _full((hid, lat)),
                    _full((1, lat)), _full((1, lat)), _full((1, lat))],
        out_specs=[_rows(EB, lat)] * 2,
        out_shape=[jax.ShapeDtypeStruct((e, lat), jnp.float32),
                   jax.ShapeDtypeStruct((e, lat), jnp.bfloat16)],
    )(edge_features, gsum, *_p6(enc_p), w1e, proc_p["b1"][None, :],
      proc_p["w2"], proc_p["b2"][None, :], proc_p["g"][None, :],
      proc_p["o"][None, :])


def _edge_step1_body(ecur, gsum, w1e, b1, w2, b2, g, o, ne_out):
    h = jnp.maximum(
        jnp.dot(ecur[...].astype(jnp.float32), w1e[...],
                preferred_element_type=jnp.float32)
        + gsum[...] + b1[...], 0.0)
    y = jnp.dot(h, w2[...], preferred_element_type=jnp.float32) + b2[...]
    m = jnp.mean(y, axis=-1, keepdims=True)
    v = jnp.mean((y - m) ** 2, axis=-1, keepdims=True)
    ne_out[...] = g[...] * (y - m) / jnp.sqrt(v + 1e-5) + o[...]


def _edge_step1(ecur, gsum, w1e, proc_p):
    e, lat = ecur.shape
    hid = proc_p["w2"].shape[0]
    grid = (e // EB,)
    return pl.pallas_call(
        _edge_step1_body,
        grid=grid,
        in_specs=[_rows(EB, lat), _rows(EB, lat), _full((lat, hid)),
                  _full((1, hid)), _full((hid, lat)), _full((1, lat)),
                  _full((1, lat)), _full((1, lat))],
        out_specs=[_rows(EB, lat)],
        out_shape=[jax.ShapeDtypeStruct((e, lat), jnp.float32)],
    )(ecur, gsum, w1e, proc_p["b1"][None, :], proc_p["w2"],
      proc_p["b2"][None, :], proc_p["g"][None, :], proc_p["o"][None, :])[0]


def _node_step0_body(nodes, agg, w1n, w1a, b1, w2, b2, g, o, ws, wr,
                     n_out, gs_out, gr_out):
    a = agg[...]
    h = jnp.maximum(
        jnp.dot(nodes[...], w1n[...], preferred_element_type=jnp.float32)
        + jnp.dot(a, w1a[...], preferred_element_type=jnp.float32)
        + b1[...], 0.0)
    y = jnp.dot(h, w2[...], preferred_element_type=jnp.float32) + b2[...]
    m = jnp.mean(y, axis=-1, keepdims=True)
    v = jnp.mean((y - m) ** 2, axis=-1, keepdims=True)
    n1 = nodes[...] + g[...] * (y - m) / jnp.sqrt(v + 1e-5) + o[...]
    n_out[...] = n1
    gs_out[...] = jnp.dot(n1, ws[...], preferred_element_type=jnp.float32)
    gr_out[...] = jnp.dot(n1, wr[...], preferred_element_type=jnp.float32)


def _node_step0(nodes, agg, w1n, w1a, proc_p, w1s, w1r):
    n, lat = nodes.shape
    hid = proc_p["w2"].shape[0]
    grid = (n // NB,)
    return pl.pallas_call(
        _node_step0_body,
        grid=grid,
        in_specs=[_rows(NB, lat), _rows(NB, lat),
                  _full((lat, hid)), _full((lat, hid)), _full((1, hid)),
                  _full((hid, lat)), _full((1, lat)), _full((1, lat)),
                  _full((1, lat)), _full((lat, lat)), _full((lat, lat))],
        out_specs=[_rows(NB, lat)] * 3,
        out_shape=[jax.ShapeDtypeStruct((n, lat), jnp.float32)] * 3,
    )(nodes, agg, w1n, w1a, proc_p["b1"][None, :], proc_p["w2"],
      proc_p["b2"][None, :], proc_p["g"][None, :], proc_p["o"][None, :],
      w1s, w1r)


def _node_step1_decode_body(nodes, agg, w1n, w1a, b1, w2, b2, g, o,
                            pw1, pb1, pw2, pb2, pg, po,
                            qw1, qb1, qw2, qb2, qg, qo,
                            lw1, lb1, lw2, lb2, lg, lo,
                            emb_out, proj_out, pred_out, logit_out):
    a = agg[...]
    h = jnp.maximum(
        jnp.dot(nodes[...], w1n[...], preferred_element_type=jnp.float32)
        + jnp.dot(a, w1a[...], preferred_element_type=jnp.float32)
        + b1[...], 0.0)
    y = jnp.dot(h, w2[...], preferred_element_type=jnp.float32) + b2[...]
    m = jnp.mean(y, axis=-1, keepdims=True)
    v = jnp.mean((y - m) ** 2, axis=-1, keepdims=True)
    n2 = nodes[...] + g[...] * (y - m) / jnp.sqrt(v + 1e-5) + o[...]
    emb_out[...] = n2
    proj = _mlp_ln_val(n2, pw1[...], pb1[...], pw2[...], pb2[...], pg[...],
                       po[...])
    proj_out[...] = proj
    pred_out[...] = _mlp_ln_val(proj, qw1[...], qb1[...], qw2[...], qb2[...],
                                qg[...], qo[...])
    logit_out[...] = _mlp_ln_val(n2, lw1[...], lb1[...], lw2[...], lb2[...],
                                 lg[...], lo[...])


def _node_step1_decode(nodes, agg, w1n, w1a, proc_p, proj_p, pred_p, log_p):
    n, lat = nodes.shape
    hid = proc_p["w2"].shape[0]
    ncls = log_p["w2"].shape[1]
    grid = (n // NB,)
    return pl.pallas_call(
        _node_step1_decode_body,
        grid=grid,
        in_specs=[_rows(NB, lat), _rows(NB, lat),
                  _full((lat, hid)), _full((lat, hid)), _full((1, hid)),
                  _full((hid, lat)), _full((1, lat)), _full((1, lat)),
                  _full((1, lat))]
                 + _p6_specs(lat, hid, lat) + _p6_specs(lat, hid, lat)
                 + _p6_specs(lat, hid, ncls),
        out_specs=[_rows(NB, lat)] * 3 + [_rows(NB, ncls)],
        out_shape=[jax.ShapeDtypeStruct((n, lat), jnp.float32)] * 3
                  + [jax.ShapeDtypeStruct((n, ncls), jnp.float32)],
    )(nodes, agg, w1n, w1a, proc_p["b1"][None, :], proc_p["w2"],
      proc_p["b2"][None, :], proc_p["g"][None, :], proc_p["o"][None, :],
      *_p6(proj_p), *_p6(pred_p), *_p6(log_p))


def _sc_mesh():
    return plsc.VectorSubcoreMesh(core_axis_name="c", subcore_axis_name="s",
                                  num_cores=SC_NC, num_subcores=SC_NS)


def _gather_sum_body(gs_hbm, gr_hbm, snd2_hbm, rcv2_hbm, out_hbm,
                     idx_s, idx_r, ra0, rb0, ro0, ra1, rb1, ro1,
                     sg0, sg1, sw0, sw1):
    nchunk = idx_s.shape[0]
    lat = ra0.shape[1]
    wid = lax.axis_index("s") * SC_NC + lax.axis_index("c")
    base0 = wid * (nchunk * SC_C)
    ra = (ra0, ra1)
    rb = (rb0, rb1)
    ro = (ro0, ro1)
    sg = (sg0, sg1)
    sw = (sw0, sw1)

    pltpu.sync_copy(snd2_hbm.at[pl.ds(wid * nchunk, nchunk)], idx_s)
    pltpu.sync_copy(rcv2_hbm.at[pl.ds(wid * nchunk, nchunk)], idx_r)

    def issue_gather(j, p):
        pltpu.async_copy(gs_hbm.at[idx_s.at[j]], ra[p], sg[p])
        pltpu.async_copy(gr_hbm.at[idx_r.at[j]], rb[p], sg[p])

    def wait_gather(j, p):
        pltpu.make_async_copy(gs_hbm.at[idx_s.at[j]], ra[p], sg[p]).wait()
        pltpu.make_async_copy(gr_hbm.at[idx_r.at[j]], rb[p], sg[p]).wait()

    def do_add(p):
        def addrow(r, c2):
            for j in range(lat // 16):
                sl = pl.ds(j * 16, 16)
                ro[p][r, sl] = ra[p][r, sl] + rb[p][r, sl]
            return c2
        lax.fori_loop(0, SC_C, addrow, 0)

    def issue_write(j, p):
        pltpu.async_copy(ro[p], out_hbm.at[pl.ds(base0 + j * SC_C, SC_C)],
                         sw[p])

    def wait_write(j, p):
        pltpu.make_async_copy(
            ro[p], out_hbm.at[pl.ds(base0 + j * SC_C, SC_C)], sw[p]).wait()

    issue_gather(0, 0)
    issue_gather(1, 1)
    for p in (0, 1):
        wait_gather(p, p)
        do_add(p)
        issue_write(p, p)
        issue_gather(p + 2, p)

    def pair(t, carry):
        for p in (0, 1):
            jj = 2 * t + p
            wait_gather(jj, p)
            wait_write(jj - 2, p)
            do_add(p)
            issue_write(jj, p)
            issue_gather(jj + 2, p)
        return carry

    lax.fori_loop(1, (nchunk - 2) // 2, pair, 0)

    for p in (0, 1):
        jj = nchunk - 2 + p
        wait_gather(jj, p)
        wait_write(jj - 2, p)
        do_add(p)
        issue_write(jj, p)
    for p in (0, 1):
        wait_write(nchunk - 2 + p, p)


def _gather_sum(gs, gr, snd2, rcv2):
    n_edges = snd2.shape[0] * snd2.shape[1]
    lat = gs.shape[1]
    nchunk = n_edges // (SC_NW * SC_C)
    f = pl.kernel(
        _gather_sum_body,
        out_type=jax.ShapeDtypeStruct((n_edges, lat), jnp.float32),
        mesh=_sc_mesh(),
        compiler_params=pltpu.CompilerParams(use_tc_tiling_on_sc=False),
        scratch_types=[
            pltpu.VMEM((nchunk, SC_C), jnp.int32),
            pltpu.VMEM((nchunk, SC_C), jnp.int32),
            pltpu.VMEM((SC_C, lat), jnp.float32),
            pltpu.VMEM((SC_C, lat), jnp.float32),
            pltpu.VMEM((SC_C, lat), jnp.float32),
            pltpu.VMEM((SC_C, lat), jnp.float32),
            pltpu.VMEM((SC_C, lat), jnp.float32),
            pltpu.VMEM((SC_C, lat), jnp.float32),
            pltpu.SemaphoreType.DMA,
            pltpu.SemaphoreType.DMA,
            pltpu.SemaphoreType.DMA,
            pltpu.SemaphoreType.DMA,
        ],
    )
    return f(gs, gr, snd2, rcv2)


def _segment_sum_body(ne_hbm, rcv2_hbm, out_hbm, shared, zbuf, idx_v,
                      rv0, rv1, sl0, sl1):
    lat = ne_hbm.shape[1]
    hc = lat // SC_NC
    nchunk = idx_v.shape[0]
    cid = lax.axis_index("c")
    sid = lax.axis_index("s")
    base0 = sid * (nchunk * SC_C)
    rv = (rv0, rv1)
    sl = (sl0, sl1)
    rows_per_sub = NPAD // SC_NS
    zero = jnp.zeros((16,), jnp.float32)

    def zrow(r, carry):
        for j in range(hc // 16):
            zbuf[r, pl.ds(j * 16, 16)] = zero
        return carry

    lax.fori_loop(0, rows_per_sub, zrow, 0)
    pltpu.sync_copy(zbuf, shared.at[pl.ds(sid * rows_per_sub, rows_per_sub)])
    pltpu.sync_copy(rcv2_hbm.at[pl.ds(sid * nchunk, nchunk)], idx_v)
    plsc.subcore_barrier()

    def issue_load(j, p):
        pltpu.async_copy(
            ne_hbm.at[pl.ds(base0 + j * SC_C, SC_C), pl.ds(cid * hc, hc)],
            rv[p], sl[p])

    def wait_load(j, p):
        pltpu.make_async_copy(
            ne_hbm.at[pl.ds(base0 + j * SC_C, SC_C), pl.ds(cid * hc, hc)],
            rv[p], sl[p]).wait()

    def scat(j, p):
        pltpu.sync_copy(rv[p], shared.at[idx_v.at[j]], add=True)

    issue_load(0, 0)
    issue_load(1, 1)
    for p in (0, 1):
        wait_load(p, p)
        scat(p, p)
        issue_load(p + 2, p)

    def pair(t, carry):
        for p in (0, 1):
            jj = 2 * t + p
            wait_load(jj, p)
            scat(jj, p)
            issue_load(jj + 2, p)
        return carry

    lax.fori_loop(1, (nchunk - 2) // 2, pair, 0)

    for p in (0, 1):
        jj = nchunk - 2 + p
        wait_load(jj, p)
        scat(jj, p)

    plsc.subcore_barrier()
    pltpu.sync_copy(shared.at[pl.ds(sid * rows_per_sub, rows_per_sub)],
                    out_hbm.at[pl.ds(sid * rows_per_sub, rows_per_sub),
                               pl.ds(cid * hc, hc)])


def _segment_sum(ne, rcv2):
    lat = ne.shape[1]
    hc = lat // SC_NC
    n_edges = ne.shape[0]
    nchunk = n_edges // (SC_NS * SC_C)
    f = pl.kernel(
        _segment_sum_body,
        out_type=jax.ShapeDtypeStruct((NPAD, lat), jnp.float32),
        mesh=_sc_mesh(),
        compiler_params=pltpu.CompilerParams(use_tc_tiling_on_sc=False),
        scratch_types=[
            pltpu.VMEM_SHARED((NPAD, hc), jnp.float32),
            pltpu.VMEM((NPAD // SC_NS, hc), jnp.float32),
            pltpu.VMEM((nchunk, SC_C), jnp.int32),
            pltpu.VMEM((SC_C, hc), jnp.float32),
            pltpu.VMEM((SC_C, hc), jnp.float32),
            pltpu.SemaphoreType.DMA,
            pltpu.SemaphoreType.DMA,
        ],
    )
    return f(ne, rcv2)


def kernel(node_features, edge_features, senders, receivers, params):
    p = params
    n_nodes = node_features.shape[0]

    def w1_split3(pp):
        lat = pp["w2"].shape[0]
        return (pp["w1"][:lat], pp["w1"][lat:2 * lat], pp["w1"][2 * lat:])

    def w1_split2(pp):
        lat = pp["w2"].shape[0]
        return (pp["w1"][:lat], pp["w1"][lat:])

    e_w1e0, e_w1s0, e_w1r0 = w1_split3(p["edge_proc_0"])
    e_w1e1, e_w1s1, e_w1r1 = w1_split3(p["edge_proc_1"])
    n_w1n0, n_w1a0 = w1_split2(p["node_proc_0"])
    n_w1n1, n_w1a1 = w1_split2(p["node_proc_1"])

    snd2 = senders.reshape(-1, SC_C)
    rcv2 = receivers.reshape(-1, SC_C)

    nodes0, gs0, gr0 = _node_encode(node_features, p["node_encoder"],
                                    e_w1s0, e_w1r0)
    gsum0 = _gather_sum(gs0, gr0, snd2, rcv2)
    ne0, e1 = _edge_step0(edge_features, gsum0, p["edge_encoder"], e_w1e0,
                          p["edge_proc_0"])
    agg0 = _segment_sum(ne0, rcv2)
    nodes1, gs1, gr1 = _node_step0(nodes0, agg0, n_w1n0, n_w1a0,
                                   p["node_proc_0"], e_w1s1, e_w1r1)
    gsum1 = _gather_sum(gs1, gr1, snd2, rcv2)
    ne1 = _edge_step1(e1, gsum1, e_w1e1, p["edge_proc_1"])
    agg1 = _segment_sum(ne1, rcv2)
    emb, proj, pred, logit = _node_step1_decode(
        nodes1, agg1, n_w1n1, n_w1a1, p["node_proc_1"],
        p["projector"], p["predictor"], p["logits_decoder"])
    return (emb, proj, pred, logit)


# speedup vs baseline: 1.4448x; 1.0753x over previous
import functools

import jax
import jax.numpy as jnp
from jax import lax
from jax.experimental import pallas as pl
from jax.experimental.pallas import tpu as pltpu
from jax.experimental.pallas import tpu_sc as plsc

N_NODES_C = 10000
N_EDGES_C = 320000
EB = 2000
NB = 2000

SC_NC = 2
SC_NS = 16
SC_NW = SC_NC * SC_NS
SC_C = 40
NPAD = 10240


def _mlp_ln_val(x, w1, b1, w2, b2, g, o):
    h = jnp.maximum(jnp.dot(x, w1, preferred_element_type=jnp.float32) + b1, 0.0)
    y = jnp.dot(h, w2, preferred_element_type=jnp.float32) + b2
    m = jnp.mean(y, axis=-1, keepdims=True)
    v = jnp.mean((y - m) ** 2, axis=-1, keepdims=True)
    return g * (y - m) / jnp.sqrt(v + 1e-5) + o


def _full(shape):
    return pl.BlockSpec(shape, lambda i: (0,) * len(shape))


def _rows(bs, d):
    return pl.BlockSpec((bs, d), lambda i: (i, 0))


def _p6(p):
    return (p["w1"], p["b1"][None, :], p["w2"], p["b2"][None, :],
            p["g"][None, :], p["o"][None, :])


def _p6_specs(in_dim, hid, out_dim):
    return [_full((in_dim, hid)), _full((1, hid)), _full((hid, out_dim)),
            _full((1, out_dim)), _full((1, out_dim)), _full((1, out_dim))]


def _node_encode_body(x, w1, b1, w2, b2, g, o, ws, wr, n_out, gs_out, gr_out):
    n = _mlp_ln_val(x[...], w1[...], b1[...], w2[...], b2[...], g[...], o[...])
    n_out[...] = n
    gs_out[...] = jnp.dot(n, ws[...], preferred_element_type=jnp.float32)
    gr_out[...] = jnp.dot(n, wr[...], preferred_element_type=jnp.float32)


def _node_encode(node_features, enc_p, w1s, w1r):
    n, d = node_features.shape
    lat = enc_p["w2"].shape[1]
    grid = (n // NB,)
    return pl.pallas_call(
        _node_encode_body,
        grid=grid,
        in_specs=[_rows(NB, d)] + _p6_specs(d, enc_p["w1"].shape[1], lat)
                 + [_full((lat, lat)), _full((lat, lat))],
        out_specs=[_rows(NB, lat)] * 3,
        out_shape=[jax.ShapeDtypeStruct((n, lat), jnp.float32)] * 3,
    )(node_features, *_p6(enc_p), w1s, w1r)


def _edge_step0_body(feat, gsum, ew1, eb1, ew2, eb2, eg, eo,
                     w1e, b1, w2, b2, g, o, ne_out, e1_out):
    e0 = _mlp_ln_val(feat[...], ew1[...], eb1[...], ew2[...], eb2[...],
                     eg[...], eo[...])
    h = jnp.maximum(
        jnp.dot(e0, w1e[...], preferred_element_type=jnp.float32)
        + gsum[...] + b1[...], 0.0)
    y = jnp.dot(h, w2[...], preferred_element_type=jnp.float32) + b2[...]
    m = jnp.mean(y, axis=-1, keepdims=True)
    v = jnp.mean((y - m) ** 2, axis=-1, keepdims=True)
    ne = g[...] * (y - m) / jnp.sqrt(v + 1e-5) + o[...]
    ne_out[...] = ne
    e1_out[...] = (e0 + ne).astype(jnp.bfloat16)


def _edge_step0(edge_features, gsum, enc_p, w1e, proc_p):
    e, d = edge_features.shape
    hid = enc_p["w1"].shape[1]
    lat = enc_p["w2"].shape[1]
    grid = (e // EB,)
    return pl.pallas_call(
        _edge_step0_body,
        grid=grid,
        in_specs=[_rows(EB, d), _rows(EB, lat)]
                 + _p6_specs(d, hid, lat)
                 + [_full((lat, hid)), _full((1, hid)), _full((hid, lat)),
                    _full((1, lat)), _full((1, lat)), _full((1, lat))],
        out_specs=[_rows(EB, lat)] * 2,
        out_shape=[jax.ShapeDtypeStruct((e, lat), jnp.float32),
                   jax.ShapeDtypeStruct((e, lat), jnp.bfloat16)],
    )(edge_features, gsum, *_p6(enc_p), w1e, proc_p["b1"][None, :],
      proc_p["w2"], proc_p["b2"][None, :], proc_p["g"][None, :],
      proc_p["o"][None, :])


def _edge_step1_body(ecur, gsum, w1e, b1, w2, b2, g, o, ne_out):
    h = jnp.maximum(
        jnp.dot(ecur[...].astype(jnp.float32), w1e[...],
                preferred_element_type=jnp.float32)
        + gsum[...] + b1[...], 0.0)
    y = jnp.dot(h, w2[...], preferred_element_type=jnp.float32) + b2[...]
    m = jnp.mean(y, axis=-1, keepdims=True)
    v = jnp.mean((y - m) ** 2, axis=-1, keepdims=True)
    ne_out[...] = g[...] * (y - m) / jnp.sqrt(v + 1e-5) + o[...]


def _edge_step1(ecur, gsum, w1e, proc_p):
    e, lat = ecur.shape
    hid = proc_p["w2"].shape[0]
    grid = (e // EB,)
    return pl.pallas_call(
        _edge_step1_body,
        grid=grid,
        in_specs=[_rows(EB, lat), _rows(EB, lat), _full((lat, hid)),
                  _full((1, hid)), _full((hid, lat)), _full((1, lat)),
                  _full((1, lat)), _full((1, lat))],
        out_specs=[_rows(EB, lat)],
        out_shape=[jax.ShapeDtypeStruct((e, lat), jnp.float32)],
    )(ecur, gsum, w1e, proc_p["b1"][None, :], proc_p["w2"],
      proc_p["b2"][None, :], proc_p["g"][None, :], proc_p["o"][None, :])[0]


def _node_step0_body(nodes, agg, w1n, w1a, b1, w2, b2, g, o, ws, wr,
                     n_out, gs_out, gr_out):
    a = agg[...]
    h = jnp.maximum(
        jnp.dot(nodes[...], w1n[...], preferred_element_type=jnp.float32)
        + jnp.dot(a, w1a[...], preferred_element_type=jnp.float32)
        + b1[...], 0.0)
    y = jnp.dot(h, w2[...], preferred_element_type=jnp.float32) + b2[...]
    m = jnp.mean(y, axis=-1, keepdims=True)
    v = jnp.mean((y - m) ** 2, axis=-1, keepdims=True)
    n1 = nodes[...] + g[...] * (y - m) / jnp.sqrt(v + 1e-5) + o[...]
    n_out[...] = n1
    gs_out[...] = jnp.dot(n1, ws[...], preferred_element_type=jnp.float32)
    gr_out[...] = jnp.dot(n1, wr[...], preferred_element_type=jnp.float32)


def _node_step0(nodes, agg, w1n, w1a, proc_p, w1s, w1r):
    n, lat = nodes.shape
    hid = proc_p["w2"].shape[0]
    grid = (n // NB,)
    return pl.pallas_call(
        _node_step0_body,
        grid=grid,
        in_specs=[_rows(NB, lat), _rows(NB, lat),
                  _full((lat, hid)), _full((lat, hid)), _full((1, hid)),
                  _full((hid, lat)), _full((1, lat)), _full((1, lat)),
                  _full((1, lat)), _full((lat, lat)), _full((lat, lat))],
        out_specs=[_rows(NB, lat)] * 3,
        out_shape=[jax.ShapeDtypeStruct((n, lat), jnp.float32)] * 3,
    )(nodes, agg, w1n, w1a, proc_p["b1"][None, :], proc_p["w2"],
      proc_p["b2"][None, :], proc_p["g"][None, :], proc_p["o"][None, :],
      w1s, w1r)


def _node_step1_decode_body(nodes, agg, w1n, w1a, b1, w2, b2, g, o,
                            pw1, pb1, pw2, pb2, pg, po,
                            qw1, qb1, qw2, qb2, qg, qo,
                            lw1, lb1, lw2, lb2, lg, lo,
                            emb_out, proj_out, pred_out, logit_out):
    a = agg[...]
    h = jnp.maximum(
        jnp.dot(nodes[...], w1n[...], preferred_element_type=jnp.float32)
        + jnp.dot(a, w1a[...], preferred_element_type=jnp.float32)
        + b1[...], 0.0)
    y = jnp.dot(h, w2[...], preferred_element_type=jnp.float32) + b2[...]
    m = jnp.mean(y, axis=-1, keepdims=True)
    v = jnp.mean((y - m) ** 2, axis=-1, keepdims=True)
    n2 = nodes[...] + g[...] * (y - m) / jnp.sqrt(v + 1e-5) + o[...]
    emb_out[...] = n2
    proj = _mlp_ln_val(n2, pw1[...], pb1[...], pw2[...], pb2[...], pg[...],
                       po[...])
    proj_out[...] = proj
    pred_out[...] = _mlp_ln_val(proj, qw1[...], qb1[...], qw2[...], qb2[...],
                                qg[...], qo[...])
    logit_out[...] = _mlp_ln_val(n2, lw1[...], lb1[...], lw2[...], lb2[...],
                                 lg[...], lo[...])


def _node_step1_decode(nodes, agg, w1n, w1a, proc_p, proj_p, pred_p, log_p):
    n, lat = nodes.shape
    hid = proc_p["w2"].shape[0]
    ncls = log_p["w2"].shape[1]
    grid = (n // NB,)
    return pl.pallas_call(
        _node_step1_decode_body,
        grid=grid,
        in_specs=[_rows(NB, lat), _rows(NB, lat),
                  _full((lat, hid)), _full((lat, hid)), _full((1, hid)),
                  _full((hid, lat)), _full((1, lat)), _full((1, lat)),
                  _full((1, lat))]
                 + _p6_specs(lat, hid, lat) + _p6_specs(lat, hid, lat)
                 + _p6_specs(lat, hid, ncls),
        out_specs=[_rows(NB, lat)] * 3 + [_rows(NB, ncls)],
        out_shape=[jax.ShapeDtypeStruct((n, lat), jnp.float32)] * 3
                  + [jax.ShapeDtypeStruct((n, ncls), jnp.float32)],
    )(nodes, agg, w1n, w1a, proc_p["b1"][None, :], proc_p["w2"],
      proc_p["b2"][None, :], proc_p["g"][None, :], proc_p["o"][None, :],
      *_p6(proj_p), *_p6(pred_p), *_p6(log_p))


def _sc_mesh():
    return plsc.VectorSubcoreMesh(core_axis_name="c", subcore_axis_name="s",
                                  num_cores=SC_NC, num_subcores=SC_NS)


def _gather_sum_body(gs_hbm, gr_hbm, snd2_hbm, rcv2_hbm, out_hbm,
                     idx_s, idx_r, ra, rb, ro, sg, sw):
    nchunk = idx_s.shape[0]
    lat = ra[0].shape[1]
    wid = lax.axis_index("s") * SC_NC + lax.axis_index("c")
    base0 = wid * (nchunk * SC_C)

    pltpu.sync_copy(snd2_hbm.at[pl.ds(wid * nchunk, nchunk)], idx_s)
    pltpu.sync_copy(rcv2_hbm.at[pl.ds(wid * nchunk, nchunk)], idx_r)

    nbuf = len(ra)

    def issue_gather(j, p):
        pltpu.async_copy(gs_hbm.at[idx_s.at[j]], ra[p], sg[p])
        pltpu.async_copy(gr_hbm.at[idx_r.at[j]], rb[p], sg[p])

    def wait_gather(j, p):
        pltpu.make_async_copy(gs_hbm.at[idx_s.at[j]], ra[p], sg[p]).wait()
        pltpu.make_async_copy(gr_hbm.at[idx_r.at[j]], rb[p], sg[p]).wait()

    def do_add(p):
        def addrow(r, c2):
            for j in range(lat // 16):
                sl = pl.ds(j * 16, 16)
                ro[p][r, sl] = ra[p][r, sl] + rb[p][r, sl]
            return c2
        lax.fori_loop(0, SC_C, addrow, 0)

    def issue_write(j, p):
        pltpu.async_copy(ro[p], out_hbm.at[pl.ds(base0 + j * SC_C, SC_C)],
                         sw[p])

    def wait_write(j, p):
        pltpu.make_async_copy(
            ro[p], out_hbm.at[pl.ds(base0 + j * SC_C, SC_C)], sw[p]).wait()

    for p in range(nbuf):
        issue_gather(p, p)

    def quad(t, carry):
        for p in range(nbuf):
            k = nbuf * t + p

            @pl.when(k < nchunk)
            def _():
                wait_gather(k, p)

                @pl.when(k >= nbuf)
                def _():
                    wait_write(k - nbuf, p)

                do_add(p)
                issue_write(k, p)

                @pl.when(k + nbuf < nchunk)
                def _():
                    issue_gather(k + nbuf, p)
        return carry

    lax.fori_loop(0, (nchunk + nbuf - 1) // nbuf, quad, 0)

    for j in range(nbuf):
        k = nchunk - nbuf + j
        wait_write(k, k % nbuf)


def _gather_sum(gs, gr, snd2, rcv2):
    n_edges = snd2.shape[0] * snd2.shape[1]
    lat = gs.shape[1]
    nchunk = n_edges // (SC_NW * SC_C)
    nbuf = 4
    f = pl.kernel(
        _gather_sum_body,
        out_type=jax.ShapeDtypeStruct((n_edges, lat), jnp.float32),
        mesh=_sc_mesh(),
        compiler_params=pltpu.CompilerParams(use_tc_tiling_on_sc=False),
        scratch_types=[
            pltpu.VMEM((nchunk, SC_C), jnp.int32),
            pltpu.VMEM((nchunk, SC_C), jnp.int32),
            [pltpu.VMEM((SC_C, lat), jnp.float32)] * nbuf,
            [pltpu.VMEM((SC_C, lat), jnp.float32)] * nbuf,
            [pltpu.VMEM((SC_C, lat), jnp.float32)] * nbuf,
            [pltpu.SemaphoreType.DMA] * nbuf,
            [pltpu.SemaphoreType.DMA] * nbuf,
        ],
    )
    return f(gs, gr, snd2, rcv2)


def _segment_sum_body(ne_hbm, rcv2_hbm, out_hbm, shared, zbuf, idx_v, rv,
                      sl, ss):
    lat = ne_hbm.shape[1]
    hc = lat // SC_NC
    nchunk, csz = idx_v.shape
    nbuf = len(rv)
    cid = lax.axis_index("c")
    sid = lax.axis_index("s")
    base0 = sid * (nchunk * csz)
    rows_per_sub = NPAD // SC_NS
    zero = jnp.zeros((16,), jnp.float32)

    def zrow(r, carry):
        for j in range(hc // 16):
            zbuf[r, pl.ds(j * 16, 16)] = zero
        return carry

    lax.fori_loop(0, rows_per_sub, zrow, 0)
    pltpu.sync_copy(zbuf, shared.at[pl.ds(sid * rows_per_sub, rows_per_sub)])
    pltpu.sync_copy(rcv2_hbm.at[pl.ds(sid * nchunk, nchunk)], idx_v)
    plsc.subcore_barrier()

    def issue_load(j, p):
        pltpu.async_copy(
            ne_hbm.at[pl.ds(base0 + j * csz, csz), pl.ds(cid * hc, hc)],
            rv[p], sl[p])

    def wait_load(j, p):
        pltpu.make_async_copy(
            ne_hbm.at[pl.ds(base0 + j * csz, csz), pl.ds(cid * hc, hc)],
            rv[p], sl[p]).wait()

    def issue_scat(j, p):
        pltpu.async_copy(rv[p], shared.at[idx_v.at[j]], ss[p], add=True)

    def wait_scat(j, p):
        pltpu.make_async_copy(rv[p], shared.at[idx_v.at[j]], ss[p]).wait()

    for p in range(nbuf):
        issue_load(p, p)

    def quad(t, carry):
        for p in range(nbuf):
            k = nbuf * t + p

            @pl.when(k < nchunk)
            def _():
                wait_load(k, p)
                issue_scat(k, p)

            km = k - (nbuf - 1)
            q = (p + 1) % nbuf

            @pl.when(jnp.logical_and(km >= 0, km < nchunk))
            def _():
                wait_scat(km, q)

                @pl.when(km + nbuf < nchunk)
                def _():
                    issue_load(km + nbuf, q)
        return carry

    lax.fori_loop(0, (nchunk + 2 * nbuf - 2) // nbuf, quad, 0)

    plsc.subcore_barrier()
    pltpu.sync_copy(shared.at[pl.ds(sid * rows_per_sub, rows_per_sub)],
                    out_hbm.at[pl.ds(sid * rows_per_sub, rows_per_sub),
                               pl.ds(cid * hc, hc)])


def _segment_sum(ne, rcv2):
    lat = ne.shape[1]
    hc = lat // SC_NC
    n_edges = ne.shape[0]
    csz = rcv2.shape[1]
    nchunk = n_edges // (SC_NS * csz)
    nbuf = 4
    f = pl.kernel(
        _segment_sum_body,
        out_type=jax.ShapeDtypeStruct((NPAD, lat), jnp.float32),
        mesh=_sc_mesh(),
        compiler_params=pltpu.CompilerParams(use_tc_tiling_on_sc=False),
        scratch_types=[
            pltpu.VMEM_SHARED((NPAD, hc), jnp.float32),
            pltpu.VMEM((NPAD // SC_NS, hc), jnp.float32),
            pltpu.VMEM((nchunk, csz), jnp.int32),
            [pltpu.VMEM((csz, hc), jnp.float32)] * nbuf,
            [pltpu.SemaphoreType.DMA] * nbuf,
            [pltpu.SemaphoreType.DMA] * nbuf,
        ],
    )
    return f(ne, rcv2)


def kernel(node_features, edge_features, senders, receivers, params):
    p = params
    n_nodes = node_features.shape[0]

    def w1_split3(pp):
        lat = pp["w2"].shape[0]
        return (pp["w1"][:lat], pp["w1"][lat:2 * lat], pp["w1"][2 * lat:])

    def w1_split2(pp):
        lat = pp["w2"].shape[0]
        return (pp["w1"][:lat], pp["w1"][lat:])

    e_w1e0, e_w1s0, e_w1r0 = w1_split3(p["edge_proc_0"])
    e_w1e1, e_w1s1, e_w1r1 = w1_split3(p["edge_proc_1"])
    n_w1n0, n_w1a0 = w1_split2(p["node_proc_0"])
    n_w1n1, n_w1a1 = w1_split2(p["node_proc_1"])

    snd2 = senders.reshape(-1, SC_C)
    rcv2 = receivers.reshape(-1, SC_C)
    rcv2s = receivers.reshape(-1, 2 * SC_C)

    nodes0, gs0, gr0 = _node_encode(node_features, p["node_encoder"],
                                    e_w1s0, e_w1r0)
    gsum0 = _gather_sum(gs0, gr0, snd2, rcv2)
    ne0, e1 = _edge_step0(edge_features, gsum0, p["edge_encoder"], e_w1e0,
                          p["edge_proc_0"])
    agg0 = _segment_sum(ne0, rcv2s)
    nodes1, gs1, gr1 = _node_step0(nodes0, agg0, n_w1n0, n_w1a0,
                                   p["node_proc_0"], e_w1s1, e_w1r1)
    gsum1 = _gather_sum(gs1, gr1, snd2, rcv2)
    ne1 = _edge_step1(e1, gsum1, e_w1e1, p["edge_proc_1"])
    agg1 = _segment_sum(ne1, rcv2s)
    emb, proj, pred, logit = _node_step1_decode(
        nodes1, agg1, n_w1n1, n_w1a1, p["node_proc_1"],
        p["projector"], p["predictor"], p["logits_decoder"])
    return (emb, proj, pred, logit)


# speedup vs baseline: 1.5215x; 1.0531x over previous
import functools

import jax
import jax.numpy as jnp
from jax import lax
from jax.experimental import pallas as pl
from jax.experimental.pallas import tpu as pltpu
from jax.experimental.pallas import tpu_sc as plsc

N_NODES_C = 10000
N_EDGES_C = 320000
EB = 4000
NB = 2000

SC_NC = 2
SC_NS = 16
SC_NW = SC_NC * SC_NS
SC_C = 40
NPAD = 10240


def _mlp_ln_val(x, w1, b1, w2, b2, g, o):
    h = jnp.maximum(jnp.dot(x, w1, preferred_element_type=jnp.float32) + b1, 0.0)
    y = jnp.dot(h, w2, preferred_element_type=jnp.float32) + b2
    m = jnp.mean(y, axis=-1, keepdims=True)
    v = jnp.mean((y - m) ** 2, axis=-1, keepdims=True)
    return g * (y - m) / jnp.sqrt(v + 1e-5) + o


def _full(shape):
    return pl.BlockSpec(shape, lambda i: (0,) * len(shape))


def _rows(bs, d):
    return pl.BlockSpec((bs, d), lambda i: (i, 0))


def _p6(p):
    return (p["w1"], p["b1"][None, :], p["w2"], p["b2"][None, :],
            p["g"][None, :], p["o"][None, :])


def _p6_specs(in_dim, hid, out_dim):
    return [_full((in_dim, hid)), _full((1, hid)), _full((hid, out_dim)),
            _full((1, out_dim)), _full((1, out_dim)), _full((1, out_dim))]


def _node_encode_body(x, w1, b1, w2, b2, g, o, ws, wr, n_out, gs_out, gr_out):
    n = _mlp_ln_val(x[...], w1[...], b1[...], w2[...], b2[...], g[...], o[...])
    n_out[...] = n
    gs_out[...] = jnp.dot(n, ws[...], preferred_element_type=jnp.float32)
    gr_out[...] = jnp.dot(n, wr[...], preferred_element_type=jnp.float32)


def _node_encode(node_features, enc_p, w1s, w1r):
    n, d = node_features.shape
    lat = enc_p["w2"].shape[1]
    grid = (n // NB,)
    return pl.pallas_call(
        _node_encode_body,
        grid=grid,
        in_specs=[_rows(NB, d)] + _p6_specs(d, enc_p["w1"].shape[1], lat)
                 + [_full((lat, lat)), _full((lat, lat))],
        out_specs=[_rows(NB, lat)] * 3,
        out_shape=[jax.ShapeDtypeStruct((n, lat), jnp.float32)] * 3,
    )(node_features, *_p6(enc_p), w1s, w1r)


def _edge_step0_body(feat, gsum, ew1, eb1, ew2, eb2, eg, eo,
                     w1e, b1, w2, b2, g, o, ne_out, e1_out):
    e0 = _mlp_ln_val(feat[...], ew1[...], eb1[...], ew2[...], eb2[...],
                     eg[...], eo[...])
    h = jnp.maximum(
        jnp.dot(e0, w1e[...], preferred_element_type=jnp.float32)
        + gsum[...] + b1[...], 0.0)
    y = jnp.dot(h, w2[...], preferred_element_type=jnp.float32) + b2[...]
    m = jnp.mean(y, axis=-1, keepdims=True)
    v = jnp.mean((y - m) ** 2, axis=-1, keepdims=True)
    ne = g[...] * (y - m) / jnp.sqrt(v + 1e-5) + o[...]
    ne_out[...] = ne
    e1_out[...] = (e0 + ne).astype(jnp.bfloat16)


def _edge_step0(edge_features, gsum, enc_p, w1e, proc_p):
    e, d = edge_features.shape
    hid = enc_p["w1"].shape[1]
    lat = enc_p["w2"].shape[1]
    grid = (e // EB,)
    return pl.pallas_call(
        _edge_step0_body,
        grid=grid,
        in_specs=[_rows(EB, d), _rows(EB, lat)]
                 + _p6_specs(d, hid, lat)
                 + [_full((lat, hid)), _full((1, hid)), _full((hid, lat)),
                    _full((1, lat)), _full((1, lat)), _full((1, lat))],
        out_specs=[_rows(EB, lat)] * 2,
        out_shape=[jax.ShapeDtypeStruct((e, lat), jnp.float32),
                   jax.ShapeDtypeStruct((e, lat), jnp.bfloat16)],
    )(edge_features, gsum, *_p6(enc_p), w1e, proc_p["b1"][None, :],
      proc_p["w2"], proc_p["b2"][None, :], proc_p["g"][None, :],
      proc_p["o"][None, :])


def _edge_step1_body(ecur, gsum, w1e, b1, w2, b2, g, o, ne_out):
    h = jnp.maximum(
        jnp.dot(ecur[...].astype(jnp.float32), w1e[...],
                preferred_element_type=jnp.float32)
        + gsum[...] + b1[...], 0.0)
    y = jnp.dot(h, w2[...], preferred_element_type=jnp.float32) + b2[...]
    m = jnp.mean(y, axis=-1, keepdims=True)
    v = jnp.mean((y - m) ** 2, axis=-1, keepdims=True)
    ne_out[...] = g[...] * (y - m) / jnp.sqrt(v + 1e-5) + o[...]


def _edge_step1(ecur, gsum, w1e, proc_p):
    e, lat = ecur.shape
    hid = proc_p["w2"].shape[0]
    grid = (e // EB,)
    return pl.pallas_call(
        _edge_step1_body,
        grid=grid,
        in_specs=[_rows(EB, lat), _rows(EB, lat), _full((lat, hid)),
                  _full((1, hid)), _full((hid, lat)), _full((1, lat)),
                  _full((1, lat)), _full((1, lat))],
        out_specs=[_rows(EB, lat)],
        out_shape=[jax.ShapeDtypeStruct((e, lat), jnp.float32)],
    )(ecur, gsum, w1e, proc_p["b1"][None, :], proc_p["w2"],
      proc_p["b2"][None, :], proc_p["g"][None, :], proc_p["o"][None, :])[0]


def _node_step0_body(nodes, agg, w1n, w1a, b1, w2, b2, g, o, ws, wr,
                     n_out, gs_out, gr_out):
    a = agg[...]
    h = jnp.maximum(
        jnp.dot(nodes[...], w1n[...], preferred_element_type=jnp.float32)
        + jnp.dot(a, w1a[...], preferred_element_type=jnp.float32)
        + b1[...], 0.0)
    y = jnp.dot(h, w2[...], preferred_element_type=jnp.float32) + b2[...]
    m = jnp.mean(y, axis=-1, keepdims=True)
    v = jnp.mean((y - m) ** 2, axis=-1, keepdims=True)
    n1 = nodes[...] + g[...] * (y - m) / jnp.sqrt(v + 1e-5) + o[...]
    n_out[...] = n1
    gs_out[...] = jnp.dot(n1, ws[...], preferred_element_type=jnp.float32)
    gr_out[...] = jnp.dot(n1, wr[...], preferred_element_type=jnp.float32)


def _node_step0(nodes, agg, w1n, w1a, proc_p, w1s, w1r):
    n, lat = nodes.shape
    hid = proc_p["w2"].shape[0]
    grid = (n // NB,)
    return pl.pallas_call(
        _node_step0_body,
        grid=grid,
        in_specs=[_rows(NB, lat), _rows(NB, lat),
                  _full((lat, hid)), _full((lat, hid)), _full((1, hid)),
                  _full((hid, lat)), _full((1, lat)), _full((1, lat)),
                  _full((1, lat)), _full((lat, lat)), _full((lat, lat))],
        out_specs=[_rows(NB, lat)] * 3,
        out_shape=[jax.ShapeDtypeStruct((n, lat), jnp.float32)] * 3,
    )(nodes, agg, w1n, w1a, proc_p["b1"][None, :], proc_p["w2"],
      proc_p["b2"][None, :], proc_p["g"][None, :], proc_p["o"][None, :],
      w1s, w1r)


def _node_step1_decode_body(nodes, agg, w1n, w1a, b1, w2, b2, g, o,
                            pw1, pb1, pw2, pb2, pg, po,
                            qw1, qb1, qw2, qb2, qg, qo,
                            lw1, lb1, lw2, lb2, lg, lo,
                            emb_out, proj_out, pred_out, logit_out):
    a = agg[...]
    h = jnp.maximum(
        jnp.dot(nodes[...], w1n[...], preferred_element_type=jnp.float32)
        + jnp.dot(a, w1a[...], preferred_element_type=jnp.float32)
        + b1[...], 0.0)
    y = jnp.dot(h, w2[...], preferred_element_type=jnp.float32) + b2[...]
    m = jnp.mean(y, axis=-1, keepdims=True)
    v = jnp.mean((y - m) ** 2, axis=-1, keepdims=True)
    n2 = nodes[...] + g[...] * (y - m) / jnp.sqrt(v + 1e-5) + o[...]
    emb_out[...] = n2
    proj = _mlp_ln_val(n2, pw1[...], pb1[...], pw2[...], pb2[...], pg[...],
                       po[...])
    proj_out[...] = proj
    pred_out[...] = _mlp_ln_val(proj, qw1[...], qb1[...], qw2[...], qb2[...],
                                qg[...], qo[...])
    logit_out[...] = _mlp_ln_val(n2, lw1[...], lb1[...], lw2[...], lb2[...],
                                 lg[...], lo[...])


def _node_step1_decode(nodes, agg, w1n, w1a, proc_p, proj_p, pred_p, log_p):
    n, lat = nodes.shape
    hid = proc_p["w2"].shape[0]
    ncls = log_p["w2"].shape[1]
    grid = (n // NB,)
    return pl.pallas_call(
        _node_step1_decode_body,
        grid=grid,
        in_specs=[_rows(NB, lat), _rows(NB, lat),
                  _full((lat, hid)), _full((lat, hid)), _full((1, hid)),
                  _full((hid, lat)), _full((1, lat)), _full((1, lat)),
                  _full((1, lat))]
                 + _p6_specs(lat, hid, lat) + _p6_specs(lat, hid, lat)
                 + _p6_specs(lat, hid, ncls),
        out_specs=[_rows(NB, lat)] * 3 + [_rows(NB, ncls)],
        out_shape=[jax.ShapeDtypeStruct((n, lat), jnp.float32)] * 3
                  + [jax.ShapeDtypeStruct((n, ncls), jnp.float32)],
    )(nodes, agg, w1n, w1a, proc_p["b1"][None, :], proc_p["w2"],
      proc_p["b2"][None, :], proc_p["g"][None, :], proc_p["o"][None, :],
      *_p6(proj_p), *_p6(pred_p), *_p6(log_p))


def _sc_mesh():
    return plsc.VectorSubcoreMesh(core_axis_name="c", subcore_axis_name="s",
                                  num_cores=SC_NC, num_subcores=SC_NS)


def _gather_sum_body(gs_hbm, gr_hbm, snd2_hbm, rcv2_hbm, out_hbm,
                     idx_s, idx_r, ra, rb, ro, sg, sw):
    nchunk = idx_s.shape[0]
    lat = ra[0].shape[1]
    wid = lax.axis_index("s") * SC_NC + lax.axis_index("c")
    base0 = wid * (nchunk * SC_C)

    pltpu.sync_copy(snd2_hbm.at[pl.ds(wid * nchunk, nchunk)], idx_s)
    pltpu.sync_copy(rcv2_hbm.at[pl.ds(wid * nchunk, nchunk)], idx_r)

    nbuf = len(ra)

    def issue_gather(j, p):
        pltpu.async_copy(gs_hbm.at[idx_s.at[j]], ra[p], sg[p])
        pltpu.async_copy(gr_hbm.at[idx_r.at[j]], rb[p], sg[p])

    def wait_gather(j, p):
        pltpu.make_async_copy(gs_hbm.at[idx_s.at[j]], ra[p], sg[p]).wait()
        pltpu.make_async_copy(gr_hbm.at[idx_r.at[j]], rb[p], sg[p]).wait()

    def do_add(p):
        def addrow(r, c2):
            for j in range(lat // 16):
                sl = pl.ds(j * 16, 16)
                ro[p][r, sl] = ra[p][r, sl] + rb[p][r, sl]
            return c2
        lax.fori_loop(0, SC_C, addrow, 0)

    def issue_write(j, p):
        pltpu.async_copy(ro[p], out_hbm.at[pl.ds(base0 + j * SC_C, SC_C)],
                         sw[p])

    def wait_write(j, p):
        pltpu.make_async_copy(
            ro[p], out_hbm.at[pl.ds(base0 + j * SC_C, SC_C)], sw[p]).wait()

    for p in range(nbuf):
        issue_gather(p, p)

    def quad(t, carry):
        for p in range(nbuf):
            k = nbuf * t + p

            @pl.when(k < nchunk)
            def _():
                wait_gather(k, p)

                @pl.when(k >= nbuf)
                def _():
                    wait_write(k - nbuf, p)

                do_add(p)
                issue_write(k, p)

                @pl.when(k + nbuf < nchunk)
                def _():
                    issue_gather(k + nbuf, p)
        return carry

    lax.fori_loop(0, (nchunk + nbuf - 1) // nbuf, quad, 0)

    for j in range(nbuf):
        k = nchunk - nbuf + j
        wait_write(k, k % nbuf)


def _gather_sum(gs, gr, snd2, rcv2):
    n_edges = snd2.shape[0] * snd2.shape[1]
    lat = gs.shape[1]
    nchunk = n_edges // (SC_NW * SC_C)
    nbuf = 4
    f = pl.kernel(
        _gather_sum_body,
        out_type=jax.ShapeDtypeStruct((n_edges, lat), jnp.float32),
        mesh=_sc_mesh(),
        compiler_params=pltpu.CompilerParams(use_tc_tiling_on_sc=False),
        scratch_types=[
            pltpu.VMEM((nchunk, SC_C), jnp.int32),
            pltpu.VMEM((nchunk, SC_C), jnp.int32),
            [pltpu.VMEM((SC_C, lat), jnp.float32)] * nbuf,
            [pltpu.VMEM((SC_C, lat), jnp.float32)] * nbuf,
            [pltpu.VMEM((SC_C, lat), jnp.float32)] * nbuf,
            [pltpu.SemaphoreType.DMA] * nbuf,
            [pltpu.SemaphoreType.DMA] * nbuf,
        ],
    )
    return f(gs, gr, snd2, rcv2)


def _segment_sum_body(ne_hbm, rcv2_hbm, out_hbm, shared, zbuf, idx_v, rv,
                      sl, ss):
    lat = ne_hbm.shape[1]
    hc = lat // SC_NC
    nchunk, csz = idx_v.shape
    nbuf = len(rv)
    cid = lax.axis_index("c")
    sid = lax.axis_index("s")
    base0 = sid * (nchunk * csz)
    rows_per_sub = NPAD // SC_NS
    zero = jnp.zeros((16,), jnp.float32)

    def zrow(r, carry):
        for j in range(hc // 16):
            zbuf[r, pl.ds(j * 16, 16)] = zero
        return carry

    lax.fori_loop(0, rows_per_sub, zrow, 0)
    pltpu.sync_copy(zbuf, shared.at[pl.ds(sid * rows_per_sub, rows_per_sub)])
    pltpu.sync_copy(rcv2_hbm.at[pl.ds(sid * nchunk, nchunk)], idx_v)
    plsc.subcore_barrier()

    def issue_load(j, p):
        pltpu.async_copy(
            ne_hbm.at[pl.ds(base0 + j * csz, csz), pl.ds(cid * hc, hc)],
            rv[p], sl[p])

    def wait_load(j, p):
        pltpu.make_async_copy(
            ne_hbm.at[pl.ds(base0 + j * csz, csz), pl.ds(cid * hc, hc)],
            rv[p], sl[p]).wait()

    def issue_scat(j, p):
        pltpu.async_copy(rv[p], shared.at[idx_v.at[j]], ss[p], add=True)

    def wait_scat(j, p):
        pltpu.make_async_copy(rv[p], shared.at[idx_v.at[j]], ss[p]).wait()

    for p in range(nbuf):
        issue_load(p, p)

    def quad(t, carry):
        for p in range(nbuf):
            k = nbuf * t + p

            @pl.when(k < nchunk)
            def _():
                wait_load(k, p)
                issue_scat(k, p)

            km = k - (nbuf - 1)
            q = (p + 1) % nbuf

            @pl.when(jnp.logical_and(km >= 0, km < nchunk))
            def _():
                wait_scat(km, q)

                @pl.when(km + nbuf < nchunk)
                def _():
                    issue_load(km + nbuf, q)
        return carry

    lax.fori_loop(0, (nchunk + 2 * nbuf - 2) // nbuf, quad, 0)

    plsc.subcore_barrier()
    pltpu.sync_copy(shared.at[pl.ds(sid * rows_per_sub, rows_per_sub)],
                    out_hbm.at[pl.ds(sid * rows_per_sub, rows_per_sub),
                               pl.ds(cid * hc, hc)])


def _segment_sum(ne, rcv2):
    lat = ne.shape[1]
    hc = lat // SC_NC
    n_edges = ne.shape[0]
    csz = rcv2.shape[1]
    nchunk = n_edges // (SC_NS * csz)
    nbuf = 4
    f = pl.kernel(
        _segment_sum_body,
        out_type=jax.ShapeDtypeStruct((NPAD, lat), jnp.float32),
        mesh=_sc_mesh(),
        compiler_params=pltpu.CompilerParams(use_tc_tiling_on_sc=False),
        scratch_types=[
            pltpu.VMEM_SHARED((NPAD, hc), jnp.float32),
            pltpu.VMEM((NPAD // SC_NS, hc), jnp.float32),
            pltpu.VMEM((nchunk, csz), jnp.int32),
            [pltpu.VMEM((csz, hc), jnp.float32)] * nbuf,
            [pltpu.SemaphoreType.DMA] * nbuf,
            [pltpu.SemaphoreType.DMA] * nbuf,
        ],
    )
    return f(ne, rcv2)


def kernel(node_features, edge_features, senders, receivers, params):
    p = params
    n_nodes = node_features.shape[0]

    def w1_split3(pp):
        lat = pp["w2"].shape[0]
        return (pp["w1"][:lat], pp["w1"][lat:2 * lat], pp["w1"][2 * lat:])

    def w1_split2(pp):
        lat = pp["w2"].shape[0]
        return (pp["w1"][:lat], pp["w1"][lat:])

    e_w1e0, e_w1s0, e_w1r0 = w1_split3(p["edge_proc_0"])
    e_w1e1, e_w1s1, e_w1r1 = w1_split3(p["edge_proc_1"])
    n_w1n0, n_w1a0 = w1_split2(p["node_proc_0"])
    n_w1n1, n_w1a1 = w1_split2(p["node_proc_1"])

    snd2 = senders.reshape(-1, SC_C)
    rcv2 = receivers.reshape(-1, SC_C)
    rcv2s = receivers.reshape(-1, 2 * SC_C)

    nodes0, gs0, gr0 = _node_encode(node_features, p["node_encoder"],
                                    e_w1s0, e_w1r0)
    gsum0 = _gather_sum(gs0, gr0, snd2, rcv2)
    ne0, e1 = _edge_step0(edge_features, gsum0, p["edge_encoder"], e_w1e0,
                          p["edge_proc_0"])
    agg0 = _segment_sum(ne0, rcv2s)
    nodes1, gs1, gr1 = _node_step0(nodes0, agg0, n_w1n0, n_w1a0,
                                   p["node_proc_0"], e_w1s1, e_w1r1)
    gsum1 = _gather_sum(gs1, gr1, snd2, rcv2)
    ne1 = _edge_step1(e1, gsum1, e_w1e1, p["edge_proc_1"])
    agg1 = _segment_sum(ne1, rcv2s)
    emb, proj, pred, logit = _node_step1_decode(
        nodes1, agg1, n_w1n1, n_w1a1, p["node_proc_1"],
        p["projector"], p["predictor"], p["logits_decoder"])
    return (emb, proj, pred, logit)


# speedup vs baseline: 1.5592x; 1.0248x over previous
import functools

import jax
import jax.numpy as jnp
from jax import lax
from jax.experimental import pallas as pl
from jax.experimental.pallas import tpu as pltpu
from jax.experimental.pallas import tpu_sc as plsc

N_NODES_C = 10000
N_EDGES_C = 320000
EB = 8000
NB = 5000

SC_NC = 2
SC_NS = 16
SC_NW = SC_NC * SC_NS
SC_C = 40
NPAD = 10240


def _mlp_ln_val(x, w1, b1, w2, b2, g, o):
    h = jnp.maximum(jnp.dot(x, w1, preferred_element_type=jnp.float32) + b1, 0.0)
    y = jnp.dot(h, w2, preferred_element_type=jnp.float32) + b2
    m = jnp.mean(y, axis=-1, keepdims=True)
    v = jnp.mean((y - m) ** 2, axis=-1, keepdims=True)
    return g * (y - m) / jnp.sqrt(v + 1e-5) + o


def _full(shape):
    return pl.BlockSpec(shape, lambda i: (0,) * len(shape))


def _rows(bs, d):
    return pl.BlockSpec((bs, d), lambda i: (i, 0))


def _p6(p):
    return (p["w1"], p["b1"][None, :], p["w2"], p["b2"][None, :],
            p["g"][None, :], p["o"][None, :])


def _p6_specs(in_dim, hid, out_dim):
    return [_full((in_dim, hid)), _full((1, hid)), _full((hid, out_dim)),
            _full((1, out_dim)), _full((1, out_dim)), _full((1, out_dim))]


def _node_encode_body(x, w1, b1, w2, b2, g, o, ws, wr, n_out, gs_out, gr_out):
    n = _mlp_ln_val(x[...], w1[...], b1[...], w2[...], b2[...], g[...], o[...])
    n_out[...] = n
    gs_out[...] = jnp.dot(n, ws[...], preferred_element_type=jnp.float32)
    gr_out[...] = jnp.dot(n, wr[...], preferred_element_type=jnp.float32)


def _node_encode(node_features, enc_p, w1s, w1r):
    n, d = node_features.shape
    lat = enc_p["w2"].shape[1]
    grid = (n // NB,)
    return pl.pallas_call(
        _node_encode_body,
        grid=grid,
        in_specs=[_rows(NB, d)] + _p6_specs(d, enc_p["w1"].shape[1], lat)
                 + [_full((lat, lat)), _full((lat, lat))],
        out_specs=[_rows(NB, lat)] * 3,
        out_shape=[jax.ShapeDtypeStruct((n, lat), jnp.float32)] * 3,
    )(node_features, *_p6(enc_p), w1s, w1r)


def _edge_step0_body(feat, gsum, ew1, eb1, ew2, eb2, eg, eo,
                     w1e, b1, w2, b2, g, o, ne_out, e1_out):
    e0 = _mlp_ln_val(feat[...], ew1[...], eb1[...], ew2[...], eb2[...],
                     eg[...], eo[...])
    h = jnp.maximum(
        jnp.dot(e0, w1e[...], preferred_element_type=jnp.float32)
        + gsum[...] + b1[...], 0.0)
    y = jnp.dot(h, w2[...], preferred_element_type=jnp.float32) + b2[...]
    m = jnp.mean(y, axis=-1, keepdims=True)
    v = jnp.mean((y - m) ** 2, axis=-1, keepdims=True)
    ne = g[...] * (y - m) / jnp.sqrt(v + 1e-5) + o[...]
    ne_out[...] = ne
    e1_out[...] = (e0 + ne).astype(jnp.bfloat16)


def _edge_step0(edge_features, gsum, enc_p, w1e, proc_p):
    e, d = edge_features.shape
    hid = enc_p["w1"].shape[1]
    lat = enc_p["w2"].shape[1]
    grid = (e // EB,)
    return pl.pallas_call(
        _edge_step0_body,
        grid=grid,
        in_specs=[_rows(EB, d), _rows(EB, lat)]
                 + _p6_specs(d, hid, lat)
                 + [_full((lat, hid)), _full((1, hid)), _full((hid, lat)),
                    _full((1, lat)), _full((1, lat)), _full((1, lat))],
        out_specs=[_rows(EB, lat)] * 2,
        out_shape=[jax.ShapeDtypeStruct((e, lat), jnp.float32),
                   jax.ShapeDtypeStruct((e, lat), jnp.bfloat16)],
    )(edge_features, gsum, *_p6(enc_p), w1e, proc_p["b1"][None, :],
      proc_p["w2"], proc_p["b2"][None, :], proc_p["g"][None, :],
      proc_p["o"][None, :])


def _edge_step1_body(ecur, gsum, w1e, b1, w2, b2, g, o, ne_out):
    h = jnp.maximum(
        jnp.dot(ecur[...].astype(jnp.float32), w1e[...],
                preferred_element_type=jnp.float32)
        + gsum[...] + b1[...], 0.0)
    y = jnp.dot(h, w2[...], preferred_element_type=jnp.float32) + b2[...]
    m = jnp.mean(y, axis=-1, keepdims=True)
    v = jnp.mean((y - m) ** 2, axis=-1, keepdims=True)
    ne_out[...] = g[...] * (y - m) / jnp.sqrt(v + 1e-5) + o[...]


def _edge_step1(ecur, gsum, w1e, proc_p):
    e, lat = ecur.shape
    hid = proc_p["w2"].shape[0]
    grid = (e // EB,)
    return pl.pallas_call(
        _edge_step1_body,
        grid=grid,
        in_specs=[_rows(EB, lat), _rows(EB, lat), _full((lat, hid)),
                  _full((1, hid)), _full((hid, lat)), _full((1, lat)),
                  _full((1, lat)), _full((1, lat))],
        out_specs=[_rows(EB, lat)],
        out_shape=[jax.ShapeDtypeStruct((e, lat), jnp.float32)],
    )(ecur, gsum, w1e, proc_p["b1"][None, :], proc_p["w2"],
      proc_p["b2"][None, :], proc_p["g"][None, :], proc_p["o"][None, :])[0]


def _node_step0_body(nodes, agg, w1n, w1a, b1, w2, b2, g, o, ws, wr,
                     n_out, gs_out, gr_out):
    a = agg[...]
    h = jnp.maximum(
        jnp.dot(nodes[...], w1n[...], preferred_element_type=jnp.float32)
        + jnp.dot(a, w1a[...], preferred_element_type=jnp.float32)
        + b1[...], 0.0)
    y = jnp.dot(h, w2[...], preferred_element_type=jnp.float32) + b2[...]
    m = jnp.mean(y, axis=-1, keepdims=True)
    v = jnp.mean((y - m) ** 2, axis=-1, keepdims=True)
    n1 = nodes[...] + g[...] * (y - m) / jnp.sqrt(v + 1e-5) + o[...]
    n_out[...] = n1
    gs_out[...] = jnp.dot(n1, ws[...], preferred_element_type=jnp.float32)
    gr_out[...] = jnp.dot(n1, wr[...], preferred_element_type=jnp.float32)


def _node_step0(nodes, agg, w1n, w1a, proc_p, w1s, w1r):
    n, lat = nodes.shape
    hid = proc_p["w2"].shape[0]
    grid = (n // NB,)
    return pl.pallas_call(
        _node_step0_body,
        grid=grid,
        in_specs=[_rows(NB, lat), _rows(NB, lat),
                  _full((lat, hid)), _full((lat, hid)), _full((1, hid)),
                  _full((hid, lat)), _full((1, lat)), _full((1, lat)),
                  _full((1, lat)), _full((lat, lat)), _full((lat, lat))],
        out_specs=[_rows(NB, lat)] * 3,
        out_shape=[jax.ShapeDtypeStruct((n, lat), jnp.float32)] * 3,
    )(nodes, agg, w1n, w1a, proc_p["b1"][None, :], proc_p["w2"],
      proc_p["b2"][None, :], proc_p["g"][None, :], proc_p["o"][None, :],
      w1s, w1r)


def _node_step1_decode_body(nodes, agg, w1n, w1a, b1, w2, b2, g, o,
                            pw1, pb1, pw2, pb2, pg, po,
                            qw1, qb1, qw2, qb2, qg, qo,
                            lw1, lb1, lw2, lb2, lg, lo,
                            emb_out, proj_out, pred_out, logit_out):
    a = agg[...]
    h = jnp.maximum(
        jnp.dot(nodes[...], w1n[...], preferred_element_type=jnp.float32)
        + jnp.dot(a, w1a[...], preferred_element_type=jnp.float32)
        + b1[...], 0.0)
    y = jnp.dot(h, w2[...], preferred_element_type=jnp.float32) + b2[...]
    m = jnp.mean(y, axis=-1, keepdims=True)
    v = jnp.mean((y - m) ** 2, axis=-1, keepdims=True)
    n2 = nodes[...] + g[...] * (y - m) / jnp.sqrt(v + 1e-5) + o[...]
    emb_out[...] = n2
    proj = _mlp_ln_val(n2, pw1[...], pb1[...], pw2[...], pb2[...], pg[...],
                       po[...])
    proj_out[...] = proj
    pred_out[...] = _mlp_ln_val(proj, qw1[...], qb1[...], qw2[...], qb2[...],
                                qg[...], qo[...])
    logit_out[...] = _mlp_ln_val(n2, lw1[...], lb1[...], lw2[...], lb2[...],
                                 lg[...], lo[...])


def _node_step1_decode(nodes, agg, w1n, w1a, proc_p, proj_p, pred_p, log_p):
    n, lat = nodes.shape
    hid = proc_p["w2"].shape[0]
    ncls = log_p["w2"].shape[1]
    grid = (n // NB,)
    return pl.pallas_call(
        _node_step1_decode_body,
        grid=grid,
        in_specs=[_rows(NB, lat), _rows(NB, lat),
                  _full((lat, hid)), _full((lat, hid)), _full((1, hid)),
                  _full((hid, lat)), _full((1, lat)), _full((1, lat)),
                  _full((1, lat))]
                 + _p6_specs(lat, hid, lat) + _p6_specs(lat, hid, lat)
                 + _p6_specs(lat, hid, ncls),
        out_specs=[_rows(NB, lat)] * 3 + [_rows(NB, ncls)],
        out_shape=[jax.ShapeDtypeStruct((n, lat), jnp.float32)] * 3
                  + [jax.ShapeDtypeStruct((n, ncls), jnp.float32)],
    )(nodes, agg, w1n, w1a, proc_p["b1"][None, :], proc_p["w2"],
      proc_p["b2"][None, :], proc_p["g"][None, :], proc_p["o"][None, :],
      *_p6(proj_p), *_p6(pred_p), *_p6(log_p))


def _sc_mesh():
    return plsc.VectorSubcoreMesh(core_axis_name="c", subcore_axis_name="s",
                                  num_cores=SC_NC, num_subcores=SC_NS)


def _gather_sum_body(gs_hbm, gr_hbm, snd2_hbm, rcv2_hbm, out_hbm,
                     idx_s, idx_r, ra, rb, ro, sg, sw):
    nchunk = idx_s.shape[0]
    lat = ra[0].shape[1]
    wid = lax.axis_index("s") * SC_NC + lax.axis_index("c")
    base0 = wid * (nchunk * SC_C)

    pltpu.sync_copy(snd2_hbm.at[pl.ds(wid * nchunk, nchunk)], idx_s)
    pltpu.sync_copy(rcv2_hbm.at[pl.ds(wid * nchunk, nchunk)], idx_r)

    nbuf = len(ra)

    def issue_gather(j, p):
        pltpu.async_copy(gs_hbm.at[idx_s.at[j]], ra[p], sg[p])
        pltpu.async_copy(gr_hbm.at[idx_r.at[j]], rb[p], sg[p])

    def wait_gather(j, p):
        pltpu.make_async_copy(gs_hbm.at[idx_s.at[j]], ra[p], sg[p]).wait()
        pltpu.make_async_copy(gr_hbm.at[idx_r.at[j]], rb[p], sg[p]).wait()

    def do_add(p):
        def addrow(r, c2):
            for j in range(lat // 16):
                sl = pl.ds(j * 16, 16)
                ro[p][r, sl] = ra[p][r, sl] + rb[p][r, sl]
            return c2
        lax.fori_loop(0, SC_C, addrow, 0)

    def issue_write(j, p):
        pltpu.async_copy(ro[p], out_hbm.at[pl.ds(base0 + j * SC_C, SC_C)],
                         sw[p])

    def wait_write(j, p):
        pltpu.make_async_copy(
            ro[p], out_hbm.at[pl.ds(base0 + j * SC_C, SC_C)], sw[p]).wait()

    for p in range(nbuf):
        issue_gather(p, p)

    def quad(t, carry):
        for p in range(nbuf):
            k = nbuf * t + p

            @pl.when(k < nchunk)
            def _():
                wait_gather(k, p)

                @pl.when(k >= nbuf)
                def _():
                    wait_write(k - nbuf, p)

                do_add(p)
                issue_write(k, p)

                @pl.when(k + nbuf < nchunk)
                def _():
                    issue_gather(k + nbuf, p)
        return carry

    lax.fori_loop(0, (nchunk + nbuf - 1) // nbuf, quad, 0)

    for j in range(nbuf):
        k = nchunk - nbuf + j
        wait_write(k, k % nbuf)


def _gather_sum(gs, gr, snd2, rcv2):
    n_edges = snd2.shape[0] * snd2.shape[1]
    lat = gs.shape[1]
    nchunk = n_edges // (SC_NW * SC_C)
    nbuf = 4
    f = pl.kernel(
        _gather_sum_body,
        out_type=jax.ShapeDtypeStruct((n_edges, lat), jnp.float32),
        mesh=_sc_mesh(),
        compiler_params=pltpu.CompilerParams(use_tc_tiling_on_sc=False),
        scratch_types=[
            pltpu.VMEM((nchunk, SC_C), jnp.int32),
            pltpu.VMEM((nchunk, SC_C), jnp.int32),
            [pltpu.VMEM((SC_C, lat), jnp.float32)] * nbuf,
            [pltpu.VMEM((SC_C, lat), jnp.float32)] * nbuf,
            [pltpu.VMEM((SC_C, lat), jnp.float32)] * nbuf,
            [pltpu.SemaphoreType.DMA] * nbuf,
            [pltpu.SemaphoreType.DMA] * nbuf,
        ],
    )
    return f(gs, gr, snd2, rcv2)


def _segment_sum_body(ne_hbm, rcv2_hbm, out_hbm, shared, zbuf, idx_v, rv,
                      sl, ss):
    lat = ne_hbm.shape[1]
    hc = lat // SC_NC
    nchunk, csz = idx_v.shape
    nbuf = len(rv)
    cid = lax.axis_index("c")
    sid = lax.axis_index("s")
    base0 = sid * (nchunk * csz)
    rows_per_sub = NPAD // SC_NS
    zero = jnp.zeros((16,), jnp.float32)

    def zrow(r, carry):
        for j in range(hc // 16):
            zbuf[r, pl.ds(j * 16, 16)] = zero
        return carry

    lax.fori_loop(0, rows_per_sub, zrow, 0)
    pltpu.sync_copy(zbuf, shared.at[pl.ds(sid * rows_per_sub, rows_per_sub)])
    pltpu.sync_copy(rcv2_hbm.at[pl.ds(sid * nchunk, nchunk)], idx_v)
    plsc.subcore_barrier()

    def issue_load(j, p):
        pltpu.async_copy(
            ne_hbm.at[pl.ds(base0 + j * csz, csz), pl.ds(cid * hc, hc)],
            rv[p], sl[p])

    def wait_load(j, p):
        pltpu.make_async_copy(
            ne_hbm.at[pl.ds(base0 + j * csz, csz), pl.ds(cid * hc, hc)],
            rv[p], sl[p]).wait()

    def issue_scat(j, p):
        pltpu.async_copy(rv[p], shared.at[idx_v.at[j]], ss[p], add=True)

    def wait_scat(j, p):
        pltpu.make_async_copy(rv[p], shared.at[idx_v.at[j]], ss[p]).wait()

    for p in range(nbuf):
        issue_load(p, p)

    def quad(t, carry):
        for p in range(nbuf):
            k = nbuf * t + p

            @pl.when(k < nchunk)
            def _():
                wait_load(k, p)
                issue_scat(k, p)

            km = k - (nbuf - 1)
            q = (p + 1) % nbuf

            @pl.when(jnp.logical_and(km >= 0, km < nchunk))
            def _():
                wait_scat(km, q)

                @pl.when(km + nbuf < nchunk)
                def _():
                    issue_load(km + nbuf, q)
        return carry

    lax.fori_loop(0, (nchunk + 2 * nbuf - 2) // nbuf, quad, 0)

    plsc.subcore_barrier()
    pltpu.sync_copy(shared.at[pl.ds(sid * rows_per_sub, rows_per_sub)],
                    out_hbm.at[pl.ds(sid * rows_per_sub, rows_per_sub),
                               pl.ds(cid * hc, hc)])


def _segment_sum(ne, rcv2):
    lat = ne.shape[1]
    hc = lat // SC_NC
    n_edges = ne.shape[0]
    csz = rcv2.shape[1]
    nchunk = n_edges // (SC_NS * csz)
    nbuf = 4
    f = pl.kernel(
        _segment_sum_body,
        out_type=jax.ShapeDtypeStruct((NPAD, lat), jnp.float32),
        mesh=_sc_mesh(),
        compiler_params=pltpu.CompilerParams(use_tc_tiling_on_sc=False),
        scratch_types=[
            pltpu.VMEM_SHARED((NPAD, hc), jnp.float32),
            pltpu.VMEM((NPAD // SC_NS, hc), jnp.float32),
            pltpu.VMEM((nchunk, csz), jnp.int32),
            [pltpu.VMEM((csz, hc), jnp.float32)] * nbuf,
            [pltpu.SemaphoreType.DMA] * nbuf,
            [pltpu.SemaphoreType.DMA] * nbuf,
        ],
    )
    return f(ne, rcv2)


def kernel(node_features, edge_features, senders, receivers, params):
    p = params
    n_nodes = node_features.shape[0]

    def w1_split3(pp):
        lat = pp["w2"].shape[0]
        return (pp["w1"][:lat], pp["w1"][lat:2 * lat], pp["w1"][2 * lat:])

    def w1_split2(pp):
        lat = pp["w2"].shape[0]
        return (pp["w1"][:lat], pp["w1"][lat:])

    e_w1e0, e_w1s0, e_w1r0 = w1_split3(p["edge_proc_0"])
    e_w1e1, e_w1s1, e_w1r1 = w1_split3(p["edge_proc_1"])
    n_w1n0, n_w1a0 = w1_split2(p["node_proc_0"])
    n_w1n1, n_w1a1 = w1_split2(p["node_proc_1"])

    snd2 = senders.reshape(-1, SC_C)
    rcv2 = receivers.reshape(-1, SC_C)
    rcv2s = receivers.reshape(-1, 2 * SC_C)

    nodes0, gs0, gr0 = _node_encode(node_features, p["node_encoder"],
                                    e_w1s0, e_w1r0)
    gsum0 = _gather_sum(gs0, gr0, snd2, rcv2)
    ne0, e1 = _edge_step0(edge_features, gsum0, p["edge_encoder"], e_w1e0,
                          p["edge_proc_0"])
    agg0 = _segment_sum(ne0, rcv2s)
    nodes1, gs1, gr1 = _node_step0(nodes0, agg0, n_w1n0, n_w1a0,
                                   p["node_proc_0"], e_w1s1, e_w1r1)
    gsum1 = _gather_sum(gs1, gr1, snd2, rcv2)
    ne1 = _edge_step1(e1, gsum1, e_w1e1, p["edge_proc_1"])
    agg1 = _segment_sum(ne1, rcv2s)
    emb, proj, pred, logit = _node_step1_decode(
        nodes1, agg1, n_w1n1, n_w1a1, p["node_proc_1"],
        p["projector"], p["predictor"], p["logits_decoder"])
    return (emb, proj, pred, logit)


# speedup vs baseline: 1.7280x; 1.1083x over previous
import functools

import jax
import jax.numpy as jnp
from jax import lax
from jax.experimental import pallas as pl
from jax.experimental.pallas import tpu as pltpu
from jax.experimental.pallas import tpu_sc as plsc

N_NODES_C = 10000
N_EDGES_C = 320000
EB = 8000
NB = 5000

SC_NC = 2
SC_NS = 16
SC_NW = SC_NC * SC_NS
SC_C = 40
NPAD = 10240


def _mlp_ln_val(x, w1, b1, w2, b2, g, o):
    h = jnp.maximum(jnp.dot(x, w1, preferred_element_type=jnp.float32) + b1, 0.0)
    y = jnp.dot(h, w2, preferred_element_type=jnp.float32) + b2
    m = jnp.mean(y, axis=-1, keepdims=True)
    v = jnp.mean((y - m) ** 2, axis=-1, keepdims=True)
    return g * (y - m) / jnp.sqrt(v + 1e-5) + o


def _full(shape):
    return pl.BlockSpec(shape, lambda i: (0,) * len(shape))


def _rows(bs, d):
    return pl.BlockSpec((bs, d), lambda i: (i, 0))


def _p6(p):
    return (p["w1"], p["b1"][None, :], p["w2"], p["b2"][None, :],
            p["g"][None, :], p["o"][None, :])


def _p6_specs(in_dim, hid, out_dim):
    return [_full((in_dim, hid)), _full((1, hid)), _full((hid, out_dim)),
            _full((1, out_dim)), _full((1, out_dim)), _full((1, out_dim))]


def _node_encode_body(x, w1, b1, w2, b2, g, o, ws, wr, n_out, gs_out, gr_out):
    n = _mlp_ln_val(x[...], w1[...], b1[...], w2[...], b2[...], g[...], o[...])
    n_out[...] = n
    gs_out[...] = jnp.dot(n, ws[...], preferred_element_type=jnp.float32)
    gr_out[...] = jnp.dot(n, wr[...], preferred_element_type=jnp.float32)


def _node_encode(node_features, enc_p, w1s, w1r):
    n, d = node_features.shape
    lat = enc_p["w2"].shape[1]
    grid = (n // NB,)
    return pl.pallas_call(
        _node_encode_body,
        grid=grid,
        in_specs=[_rows(NB, d)] + _p6_specs(d, enc_p["w1"].shape[1], lat)
                 + [_full((lat, lat)), _full((lat, lat))],
        out_specs=[_rows(NB, lat)] * 3,
        out_shape=[jax.ShapeDtypeStruct((n, lat), jnp.float32)] * 3,
    )(node_features, *_p6(enc_p), w1s, w1r)


def _edge_step0_body(feat, gsum, ew1, eb1, ew2, eb2, eg, eo,
                     w1e, b1, w2, b2, g, o, ne_out, e1_out):
    e0 = _mlp_ln_val(feat[...], ew1[...], eb1[...], ew2[...], eb2[...],
                     eg[...], eo[...])
    h = jnp.maximum(
        jnp.dot(e0, w1e[...], preferred_element_type=jnp.float32)
        + gsum[...] + b1[...], 0.0)
    y = jnp.dot(h, w2[...], preferred_element_type=jnp.float32) + b2[...]
    m = jnp.mean(y, axis=-1, keepdims=True)
    v = jnp.mean((y - m) ** 2, axis=-1, keepdims=True)
    ne = g[...] * (y - m) / jnp.sqrt(v + 1e-5) + o[...]
    ne_out[...] = ne
    e1_out[...] = (e0 + ne).astype(jnp.bfloat16)


def _edge_step0(edge_features, gsum, enc_p, w1e, proc_p):
    e, d = edge_features.shape
    hid = enc_p["w1"].shape[1]
    lat = enc_p["w2"].shape[1]
    grid = (e // EB,)
    return pl.pallas_call(
        _edge_step0_body,
        grid=grid,
        in_specs=[_rows(EB, d), _rows(EB, lat)]
                 + _p6_specs(d, hid, lat)
                 + [_full((lat, hid)), _full((1, hid)), _full((hid, lat)),
                    _full((1, lat)), _full((1, lat)), _full((1, lat))],
        out_specs=[_rows(EB, lat)] * 2,
        out_shape=[jax.ShapeDtypeStruct((e, lat), jnp.float32),
                   jax.ShapeDtypeStruct((e, lat), jnp.bfloat16)],
    )(edge_features, gsum, *_p6(enc_p), w1e, proc_p["b1"][None, :],
      proc_p["w2"], proc_p["b2"][None, :], proc_p["g"][None, :],
      proc_p["o"][None, :])


def _edge_step1_body(ecur, gsum, w1e, b1, w2, b2, g, o, ne_out):
    h = jnp.maximum(
        jnp.dot(ecur[...].astype(jnp.float32), w1e[...],
                preferred_element_type=jnp.float32)
        + gsum[...] + b1[...], 0.0)
    y = jnp.dot(h, w2[...], preferred_element_type=jnp.float32) + b2[...]
    m = jnp.mean(y, axis=-1, keepdims=True)
    v = jnp.mean((y - m) ** 2, axis=-1, keepdims=True)
    ne_out[...] = g[...] * (y - m) / jnp.sqrt(v + 1e-5) + o[...]


def _edge_step1(ecur, gsum, w1e, proc_p):
    e, lat = ecur.shape
    hid = proc_p["w2"].shape[0]
    grid = (e // EB,)
    return pl.pallas_call(
        _edge_step1_body,
        grid=grid,
        in_specs=[_rows(EB, lat), _rows(EB, lat), _full((lat, hid)),
                  _full((1, hid)), _full((hid, lat)), _full((1, lat)),
                  _full((1, lat)), _full((1, lat))],
        out_specs=[_rows(EB, lat)],
        out_shape=[jax.ShapeDtypeStruct((e, lat), jnp.float32)],
    )(ecur, gsum, w1e, proc_p["b1"][None, :], proc_p["w2"],
      proc_p["b2"][None, :], proc_p["g"][None, :], proc_p["o"][None, :])[0]


def _node_step0_body(nodes, a0, a1, w1n, w1a, b1, w2, b2, g, o, ws, wr,
                     n_out, gs_out, gr_out):
    a = a0[...] + a1[...]
    h = jnp.maximum(
        jnp.dot(nodes[...], w1n[...], preferred_element_type=jnp.float32)
        + jnp.dot(a, w1a[...], preferred_element_type=jnp.float32)
        + b1[...], 0.0)
    y = jnp.dot(h, w2[...], preferred_element_type=jnp.float32) + b2[...]
    m = jnp.mean(y, axis=-1, keepdims=True)
    v = jnp.mean((y - m) ** 2, axis=-1, keepdims=True)
    n1 = nodes[...] + g[...] * (y - m) / jnp.sqrt(v + 1e-5) + o[...]
    n_out[...] = n1
    gs_out[...] = jnp.dot(n1, ws[...], preferred_element_type=jnp.float32)
    gr_out[...] = jnp.dot(n1, wr[...], preferred_element_type=jnp.float32)


def _node_step0(nodes, a0, a1, w1n, w1a, proc_p, w1s, w1r):
    n, lat = nodes.shape
    hid = proc_p["w2"].shape[0]
    grid = (n // NB,)
    return pl.pallas_call(
        _node_step0_body,
        grid=grid,
        in_specs=[_rows(NB, lat), _rows(NB, lat), _rows(NB, lat),
                  _full((lat, hid)), _full((lat, hid)), _full((1, hid)),
                  _full((hid, lat)), _full((1, lat)), _full((1, lat)),
                  _full((1, lat)), _full((lat, lat)), _full((lat, lat))],
        out_specs=[_rows(NB, lat)] * 3,
        out_shape=[jax.ShapeDtypeStruct((n, lat), jnp.float32)] * 3,
    )(nodes, a0, a1, w1n, w1a, proc_p["b1"][None, :], proc_p["w2"],
      proc_p["b2"][None, :], proc_p["g"][None, :], proc_p["o"][None, :],
      w1s, w1r)


def _node_step1_decode_body(nodes, a0, a1, w1n, w1a, b1, w2, b2, g, o,
                            pw1, pb1, pw2, pb2, pg, po,
                            qw1, qb1, qw2, qb2, qg, qo,
                            lw1, lb1, lw2, lb2, lg, lo,
                            emb_out, proj_out, pred_out, logit_out):
    a = a0[...] + a1[...]
    h = jnp.maximum(
        jnp.dot(nodes[...], w1n[...], preferred_element_type=jnp.float32)
        + jnp.dot(a, w1a[...], preferred_element_type=jnp.float32)
        + b1[...], 0.0)
    y = jnp.dot(h, w2[...], preferred_element_type=jnp.float32) + b2[...]
    m = jnp.mean(y, axis=-1, keepdims=True)
    v = jnp.mean((y - m) ** 2, axis=-1, keepdims=True)
    n2 = nodes[...] + g[...] * (y - m) / jnp.sqrt(v + 1e-5) + o[...]
    emb_out[...] = n2
    proj = _mlp_ln_val(n2, pw1[...], pb1[...], pw2[...], pb2[...], pg[...],
                       po[...])
    proj_out[...] = proj
    pred_out[...] = _mlp_ln_val(proj, qw1[...], qb1[...], qw2[...], qb2[...],
                                qg[...], qo[...])
    logit_out[...] = _mlp_ln_val(n2, lw1[...], lb1[...], lw2[...], lb2[...],
                                 lg[...], lo[...])


def _node_step1_decode(nodes, a0, a1, w1n, w1a, proc_p, proj_p, pred_p,
                       log_p):
    n, lat = nodes.shape
    hid = proc_p["w2"].shape[0]
    ncls = log_p["w2"].shape[1]
    grid = (n // NB,)
    return pl.pallas_call(
        _node_step1_decode_body,
        grid=grid,
        in_specs=[_rows(NB, lat), _rows(NB, lat), _rows(NB, lat),
                  _full((lat, hid)), _full((lat, hid)), _full((1, hid)),
                  _full((hid, lat)), _full((1, lat)), _full((1, lat)),
                  _full((1, lat))]
                 + _p6_specs(lat, hid, lat) + _p6_specs(lat, hid, lat)
                 + _p6_specs(lat, hid, ncls),
        out_specs=[_rows(NB, lat)] * 3 + [_rows(NB, ncls)],
        out_shape=[jax.ShapeDtypeStruct((n, lat), jnp.float32)] * 3
                  + [jax.ShapeDtypeStruct((n, ncls), jnp.float32)],
    )(nodes, a0, a1, w1n, w1a, proc_p["b1"][None, :], proc_p["w2"],
      proc_p["b2"][None, :], proc_p["g"][None, :], proc_p["o"][None, :],
      *_p6(proj_p), *_p6(pred_p), *_p6(log_p))


def _sc_mesh():
    return plsc.VectorSubcoreMesh(core_axis_name="c", subcore_axis_name="s",
                                  num_cores=SC_NC, num_subcores=SC_NS)


def _gather_sum_body(gs_hbm, gr_hbm, snd2_hbm, rcv2_hbm, out_hbm,
                     idx_s, idx_r, ra, rb, ro, sg, sw):
    nchunk = idx_s.shape[0]
    lat = ra[0].shape[1]
    wid = lax.axis_index("s") * SC_NC + lax.axis_index("c")
    base0 = wid * (nchunk * SC_C)

    pltpu.sync_copy(snd2_hbm.at[pl.ds(wid * nchunk, nchunk)], idx_s)
    pltpu.sync_copy(rcv2_hbm.at[pl.ds(wid * nchunk, nchunk)], idx_r)

    nbuf = len(ra)

    def issue_gather(j, p):
        pltpu.async_copy(gs_hbm.at[idx_s.at[j]], ra[p], sg[p])
        pltpu.async_copy(gr_hbm.at[idx_r.at[j]], rb[p], sg[p])

    def wait_gather(j, p):
        pltpu.make_async_copy(gs_hbm.at[idx_s.at[j]], ra[p], sg[p]).wait()
        pltpu.make_async_copy(gr_hbm.at[idx_r.at[j]], rb[p], sg[p]).wait()

    def do_add(p):
        def addrow(r, c2):
            for j in range(lat // 16):
                sl = pl.ds(j * 16, 16)
                ro[p][r, sl] = ra[p][r, sl] + rb[p][r, sl]
            return c2
        lax.fori_loop(0, SC_C, addrow, 0)

    def issue_write(j, p):
        pltpu.async_copy(ro[p], out_hbm.at[pl.ds(base0 + j * SC_C, SC_C)],
                         sw[p])

    def wait_write(j, p):
        pltpu.make_async_copy(
            ro[p], out_hbm.at[pl.ds(base0 + j * SC_C, SC_C)], sw[p]).wait()

    for p in range(nbuf):
        issue_gather(p, p)

    def quad(t, carry):
        for p in range(nbuf):
            k = nbuf * t + p

            @pl.when(k < nchunk)
            def _():
                wait_gather(k, p)

                @pl.when(k >= nbuf)
                def _():
                    wait_write(k - nbuf, p)

                do_add(p)
                issue_write(k, p)

                @pl.when(k + nbuf < nchunk)
                def _():
                    issue_gather(k + nbuf, p)
        return carry

    lax.fori_loop(0, (nchunk + nbuf - 1) // nbuf, quad, 0)

    for j in range(nbuf):
        k = nchunk - nbuf + j
        wait_write(k, k % nbuf)


def _gather_sum(gs, gr, snd2, rcv2):
    n_edges = snd2.shape[0] * snd2.shape[1]
    lat = gs.shape[1]
    nchunk = n_edges // (SC_NW * SC_C)
    nbuf = 4
    f = pl.kernel(
        _gather_sum_body,
        out_type=jax.ShapeDtypeStruct((n_edges, lat), jnp.float32),
        mesh=_sc_mesh(),
        compiler_params=pltpu.CompilerParams(use_tc_tiling_on_sc=False),
        scratch_types=[
            pltpu.VMEM((nchunk, SC_C), jnp.int32),
            pltpu.VMEM((nchunk, SC_C), jnp.int32),
            [pltpu.VMEM((SC_C, lat), jnp.float32)] * nbuf,
            [pltpu.VMEM((SC_C, lat), jnp.float32)] * nbuf,
            [pltpu.VMEM((SC_C, lat), jnp.float32)] * nbuf,
            [pltpu.SemaphoreType.DMA] * nbuf,
            [pltpu.SemaphoreType.DMA] * nbuf,
        ],
    )
    return f(gs, gr, snd2, rcv2)


def _segment_sum_body(ne_hbm, rcv2_hbm, out_hbm, shared, zbuf, idx_v, rv,
                      sl, ss):
    lat = ne_hbm.shape[1]
    hc = lat // SC_NC
    nchunk, csz = idx_v.shape
    nbuf = len(rv)
    cid = lax.axis_index("c")
    sid = lax.axis_index("s")
    base0 = sid * (nchunk * csz)
    rows_per_sub = NPAD // SC_NS
    zero = jnp.zeros((16,), jnp.float32)

    def zrow(r, carry):
        for j in range(hc // 16):
            zbuf[r, pl.ds(j * 16, 16)] = zero
        return carry

    lax.fori_loop(0, rows_per_sub, zrow, 0)
    pltpu.sync_copy(zbuf, shared.at[pl.ds(sid * rows_per_sub, rows_per_sub)])
    pltpu.sync_copy(rcv2_hbm.at[pl.ds(sid * nchunk, nchunk)], idx_v)
    plsc.subcore_barrier()

    def issue_load(j, p):
        pltpu.async_copy(
            ne_hbm.at[pl.ds(base0 + j * csz, csz), pl.ds(cid * hc, hc)],
            rv[p], sl[p])

    def wait_load(j, p):
        pltpu.make_async_copy(
            ne_hbm.at[pl.ds(base0 + j * csz, csz), pl.ds(cid * hc, hc)],
            rv[p], sl[p]).wait()

    def issue_scat(j, p):
        pltpu.async_copy(rv[p], shared.at[idx_v.at[j]], ss[p], add=True)

    def wait_scat(j, p):
        pltpu.make_async_copy(rv[p], shared.at[idx_v.at[j]], ss[p]).wait()

    for p in range(nbuf):
        issue_load(p, p)

    def quad(t, carry):
        for p in range(nbuf):
            k = nbuf * t + p

            @pl.when(k < nchunk)
            def _():
                wait_load(k, p)
                issue_scat(k, p)

            km = k - (nbuf - 1)
            q = (p + 1) % nbuf

            @pl.when(jnp.logical_and(km >= 0, km < nchunk))
            def _():
                wait_scat(km, q)

                @pl.when(km + nbuf < nchunk)
                def _():
                    issue_load(km + nbuf, q)
        return carry

    lax.fori_loop(0, (nchunk + 2 * nbuf - 2) // nbuf, quad, 0)

    plsc.subcore_barrier()
    pltpu.sync_copy(shared.at[pl.ds(sid * rows_per_sub, rows_per_sub)],
                    out_hbm.at[pl.ds(sid * rows_per_sub, rows_per_sub),
                               pl.ds(cid * hc, hc)])


def _segment_sum(ne, rcv2):
    lat = ne.shape[1]
    hc = lat // SC_NC
    n_edges = ne.shape[0]
    csz = rcv2.shape[1]
    nchunk = n_edges // (SC_NS * csz)
    nbuf = 4
    f = pl.kernel(
        _segment_sum_body,
        out_type=jax.ShapeDtypeStruct((NPAD, lat), jnp.float32),
        mesh=_sc_mesh(),
        compiler_params=pltpu.CompilerParams(use_tc_tiling_on_sc=False),
        scratch_types=[
            pltpu.VMEM_SHARED((NPAD, hc), jnp.float32),
            pltpu.VMEM((NPAD // SC_NS, hc), jnp.float32),
            pltpu.VMEM((nchunk, csz), jnp.int32),
            [pltpu.VMEM((csz, hc), jnp.float32)] * nbuf,
            [pltpu.SemaphoreType.DMA] * nbuf,
            [pltpu.SemaphoreType.DMA] * nbuf,
        ],
    )
    return f(ne, rcv2)


def kernel(node_features, edge_features, senders, receivers, params):
    p = params
    n_nodes = node_features.shape[0]

    def w1_split3(pp):
        lat = pp["w2"].shape[0]
        return (pp["w1"][:lat], pp["w1"][lat:2 * lat], pp["w1"][2 * lat:])

    def w1_split2(pp):
        lat = pp["w2"].shape[0]
        return (pp["w1"][:lat], pp["w1"][lat:])

    e_w1e0, e_w1s0, e_w1r0 = w1_split3(p["edge_proc_0"])
    e_w1e1, e_w1s1, e_w1r1 = w1_split3(p["edge_proc_1"])
    n_w1n0, n_w1a0 = w1_split2(p["node_proc_0"])
    n_w1n1, n_w1a1 = w1_split2(p["node_proc_1"])

    n_edges = senders.shape[0]
    he = n_edges // 2
    snd2 = senders.reshape(-1, SC_C)
    rcv2 = receivers.reshape(-1, SC_C)
    rcv2s = receivers.reshape(-1, 2 * SC_C)
    hs2 = snd2.shape[0] // 2
    hs2s = rcv2s.shape[0] // 2
    snd2_h = (snd2[:hs2], snd2[hs2:])
    rcv2_h = (rcv2[:hs2], rcv2[hs2:])
    rcv2s_h = (rcv2s[:hs2s], rcv2s[hs2s:])
    feat_h = (edge_features[:he], edge_features[he:])

    nodes0, gs0, gr0 = _node_encode(node_features, p["node_encoder"],
                                    e_w1s0, e_w1r0)
    gsum0_a = _gather_sum(gs0, gr0, snd2_h[0], rcv2_h[0])
    gsum0_b = _gather_sum(gs0, gr0, snd2_h[1], rcv2_h[1])
    ne0_a, e1_a = _edge_step0(feat_h[0], gsum0_a, p["edge_encoder"], e_w1e0,
                              p["edge_proc_0"])
    agg0_a = _segment_sum(ne0_a, rcv2s_h[0])
    ne0_b, e1_b = _edge_step0(feat_h[1], gsum0_b, p["edge_encoder"], e_w1e0,
                              p["edge_proc_0"])
    agg0_b = _segment_sum(ne0_b, rcv2s_h[1])
    nodes1, gs1, gr1 = _node_step0(nodes0, agg0_a, agg0_b, n_w1n0, n_w1a0,
                                   p["node_proc_0"], e_w1s1, e_w1r1)
    gsum1_a = _gather_sum(gs1, gr1, snd2_h[0], rcv2_h[0])
    gsum1_b = _gather_sum(gs1, gr1, snd2_h[1], rcv2_h[1])
    ne1_a = _edge_step1(e1_a, gsum1_a, e_w1e1, p["edge_proc_1"])
    agg1_a = _segment_sum(ne1_a, rcv2s_h[0])
    ne1_b = _edge_step1(e1_b, gsum1_b, e_w1e1, p["edge_proc_1"])
    agg1_b = _segment_sum(ne1_b, rcv2s_h[1])
    emb, proj, pred, logit = _node_step1_decode(
        nodes1, agg1_a, agg1_b, n_w1n1, n_w1a1, p["node_proc_1"],
        p["projector"], p["predictor"], p["logits_decoder"])
    return (emb, proj, pred, logit)
